# Initial kernel scaffold; baseline (speedup 1.0000x reference)
import functools

import jax
import jax.numpy as jnp
from jax import lax
from jax.experimental import pallas as pl
from jax.experimental.pallas import tpu as pltpu
from jax.experimental.pallas import tpu_sc as plsc

F32 = jnp.float32
I32 = jnp.int32

TW = 24
N = 10000
XE = 32
LOCE = 12
EH = 16
GH = 64
PS = 6
G = 16
E = 160000
NW = 32
CHUNK = 128
EPW = 5120
EPAD = NW * EPW
NCH = EPW // CHUNK
NPAD = 10240
RPT = NPAD // 16
FW = 80

ROWS = 1250
GRID = 8


def _k1_body(x_ref, loc_ref, wp_ref, wih_ref, whh_ref, b_ref, locw_ref,
             locb_ref, h_ref, w_ref, gx_ref):
    i = pl.program_id(0)
    h = jnp.zeros((ROWS, XE), F32)
    c = jnp.zeros((ROWS, XE), F32)
    wih = wih_ref[...]
    whh = whh_ref[...]
    b = b_ref[...]
    dn = (((0,), (0,)), ((), ()))
    for t in range(TW):
        xt = x_ref[t * 8:(t + 1) * 8, :]
        g = (lax.dot_general(xt, wih, dn, preferred_element_type=F32)
             + jnp.dot(h, whh, preferred_element_type=F32) + b)
        ig = jax.nn.sigmoid(g[:, :XE])
        fg = jax.nn.sigmoid(g[:, XE:2 * XE])
        gg = jnp.tanh(g[:, 2 * XE:3 * XE])
        og = jax.nn.sigmoid(g[:, 3 * XE:])
        c = fg * c + ig * gg
        h = og * jnp.tanh(c)
    h_ref[...] = h
    wp = wp_ref[...]
    ex = jnp.exp(wp - jnp.max(wp, axis=1, keepdims=True))
    w = ex / jnp.sum(ex, axis=1, keepdims=True)
    w_ref[...] = w
    loce = (lax.dot_general(loc_ref[...], locw_ref[...], dn,
                            preferred_element_type=F32) + locb_ref[...])
    xloc = jnp.concatenate([h, loce], axis=1)
    gxp = lax.dot_general(w, xloc, dn, preferred_element_type=F32)

    @pl.when(i == 0)
    def _():
        gx_ref[...] = jnp.zeros_like(gx_ref)

    gx_ref[...] += gxp


def _k1(xr, locr, wp, wih_t, whh_t, b2, locw_t, locb2):
    full = lambda s: pl.BlockSpec(s, lambda i: (0, 0))
    return pl.pallas_call(
        _k1_body,
        grid=(GRID,),
        in_specs=[
            pl.BlockSpec((TW * 8, ROWS), lambda i: (0, i)),
            pl.BlockSpec((2, ROWS), lambda i: (0, i)),
            pl.BlockSpec((ROWS, G), lambda i: (i, 0)),
            full((8, 4 * XE)),
            full((XE, 4 * XE)),
            full((1, 4 * XE)),
            full((2, LOCE)),
            full((1, LOCE)),
        ],
        out_specs=[
            pl.BlockSpec((ROWS, XE), lambda i: (i, 0)),
            pl.BlockSpec((ROWS, G), lambda i: (i, 0)),
            pl.BlockSpec((G, XE + LOCE), lambda i: (0, 0)),
        ],
        out_shape=[
            jax.ShapeDtypeStruct((N, XE), F32),
            jax.ShapeDtypeStruct((N, G), F32),
            jax.ShapeDtypeStruct((G, XE + LOCE), F32),
        ],
    )(xr, locr, wp, wih_t, whh_t, b2, locw_t, locb2)


def _k2_body(gx_ref, u_ref, ue1_ref, ue2_ref, ue3_ref, si_ref, sj_ref, d_ref,
             ew_t_ref, eb_ref, a1_ref, ab1_ref, a2_ref, ab2_ref, c1_ref,
             cb1_ref, c2_ref, cb2_ref, out_ref):
    gx = gx_ref[...]
    si = si_ref[...]
    sj = sj_ref[...]
    dsel = d_ref[...]
    u0 = u_ref[0]
    u1 = u_ref[1]
    u2 = u_ref[2]
    ue = jnp.concatenate([
        ue1_ref[pl.ds(u0, 1), :],
        ue2_ref[pl.ds(u1, 1), :],
        ue3_ref[pl.ds(u2, 1), :],
    ], axis=1)
    gi = jnp.dot(si, gx, preferred_element_type=F32)
    gj = jnp.dot(sj, gx, preferred_element_type=F32)
    gin = jnp.concatenate(
        [gi, gj, jnp.broadcast_to(ue, (256, 12))], axis=1)
    ge = jnp.maximum(
        jnp.dot(gin, ew_t_ref[...], preferred_element_type=F32)
        + eb_ref[...], 0.0)
    dn = (((0,), (0,)), ((), ()))

    def node_layer(xg, w1t, b1, w2t, b2):
        xi = jnp.dot(si, xg, preferred_element_type=F32)
        m = jnp.maximum(
            jnp.dot(jnp.concatenate([xi, ge], axis=1), w1t,
                    preferred_element_type=F32) + b1, 0.0)
        ssum = lax.dot_general(sj, m, dn, preferred_element_type=F32)
        diag = jnp.dot(dsel, m, preferred_element_type=F32)
        mean = (ssum - diag) * (1.0 / 15.0)
        return jnp.maximum(
            jnp.dot(jnp.concatenate([xg, mean], axis=1), w2t,
                    preferred_element_type=F32) + b2, 0.0)

    g1 = node_layer(gx, a1_ref[...], ab1_ref[...], a2_ref[...], ab2_ref[...])
    g2 = node_layer(g1, c1_ref[...], cb1_ref[...], c2_ref[...], cb2_ref[...])
    out_ref[...] = g2


def _k2(gx, u, ue1, ue2, ue3, si, sj, dsel, ew_t, eb, a1, ab1, a2, ab2, c1,
        cb1, c2, cb2):
    args = (gx, u, ue1, ue2, ue3, si, sj, dsel, ew_t, eb, a1, ab1, a2, ab2,
            c1, cb1, c2, cb2)
    in_specs = [pl.BlockSpec(a.shape, functools.partial(lambda n: (0,) * n,
                                                        a.ndim))
                for a in args]
    in_specs[1] = pl.BlockSpec(memory_space=pltpu.SMEM)
    return pl.pallas_call(
        _k2_body,
        in_specs=in_specs,
        out_specs=pl.BlockSpec((G, GH), lambda: (0, 0)),
        out_shape=jax.ShapeDtypeStruct((G, GH), F32),
    )(*args)


def _k3_body(h_ref, w_ref, gx2_ref, w1t_ref, b1_ref, nx_ref, y_ref):
    h = h_ref[...]
    gnew = jnp.dot(w_ref[...], gx2_ref[...], preferred_element_type=F32)
    nx = jnp.concatenate([h, gnew], axis=1)
    nx_ref[...] = nx
    y = jnp.dot(nx, w1t_ref[...], preferred_element_type=F32) + b1_ref[...]
    y_ref[...] = jnp.concatenate(
        [y, jnp.ones((ROWS, 1), F32), jnp.zeros((ROWS, FW - GH - 1), F32)],
        axis=1)


def _k3(h, w, gx2, w1t, b1):
    full = lambda s: pl.BlockSpec(s, lambda i: (0, 0))
    return pl.pallas_call(
        _k3_body,
        grid=(GRID,),
        in_specs=[
            pl.BlockSpec((ROWS, XE), lambda i: (i, 0)),
            pl.BlockSpec((ROWS, G), lambda i: (i, 0)),
            full((G, GH)),
            full((XE + GH, GH)),
            full((1, GH)),
        ],
        out_specs=[
            pl.BlockSpec((ROWS, XE + GH), lambda i: (i, 0)),
            pl.BlockSpec((ROWS, FW), lambda i: (i, 0)),
        ],
        out_shape=[
            jax.ShapeDtypeStruct((N, XE + GH), F32),
            jax.ShapeDtypeStruct((N, FW), F32),
        ],
    )(h, w, gx2, w1t, b1)


def _sc_body(y_hbm, row_hbm, col_hbm, ew_hbm, wl_hbm, z_hbm, out_hbm,
             row_v, col_v, ew_v, msg_v, wl_v, sums, sem):
    cid = lax.axis_index("c")
    sid = lax.axis_index("s")
    wid = sid * 2 + cid
    pltpu.sync_copy(wl_hbm, wl_v)
    pltpu.sync_copy(z_hbm, sums.at[pl.ds(sid * RPT, RPT)])
    plsc.subcore_barrier()
    wl = [wl_v[pl.ds(j * 16, 16)] for j in range(4)]

    def chunk_body(cn, carry):
        base = wid * EPW + cn * CHUNK
        pltpu.sync_copy(row_hbm.at[pl.ds(base, CHUNK)], row_v)
        pltpu.sync_copy(col_hbm.at[pl.ds(base, CHUNK)], col_v)
        pltpu.sync_copy(ew_hbm.at[pl.ds(base, CHUNK)], ew_v)
        pltpu.async_copy(y_hbm.at[row_v], msg_v, sem).wait()

        def edge_body(e, c2):
            ewb = plsc.load_gather(ew_v, [jnp.full((16,), 0, I32) + e])
            for j in range(4):
                v = msg_v[e, pl.ds(j * 16, 16)]
                msg_v[e, pl.ds(j * 16, 16)] = jnp.maximum(
                    v + ewb * wl[j], 0.0)
            return c2

        lax.fori_loop(0, CHUNK, edge_body, 0)
        pltpu.sync_copy(msg_v, sums.at[col_v], add=True)
        return carry

    lax.fori_loop(0, NCH, chunk_body, 0)
    plsc.subcore_barrier()
    pltpu.sync_copy(sums.at[pl.ds(sid * RPT, RPT)],
                    out_hbm.at[cid, pl.ds(sid * RPT, RPT)])


def _edge_partials(y_ext, rowp, colp, ewp, wl_ext, zrows):
    mesh = plsc.VectorSubcoreMesh(core_axis_name="c", subcore_axis_name="s")
    k = pl.kernel(
        _sc_body,
        mesh=mesh,
        out_type=jax.ShapeDtypeStruct((2, NPAD, FW), F32),
        scratch_types=[
            pltpu.VMEM((CHUNK,), I32),
            pltpu.VMEM((CHUNK,), I32),
            pltpu.VMEM((CHUNK,), F32),
            pltpu.VMEM((CHUNK, FW), F32),
            pltpu.VMEM((FW,), F32),
            pltpu.VMEM_SHARED((NPAD, FW), F32),
            pltpu.SemaphoreType.DMA,
        ],
    )
    return k(y_ext, rowp, colp, ewp, wl_ext, zrows)


def _k4_body(pp_ref, nx_ref, w2t_ref, b2_ref, w1t_ref, b1_ref, h0_ref, y_ref):
    s = pp_ref[0] + pp_ref[1]
    mean = s[:, :GH] / jnp.maximum(s[:, GH:GH + 1], 1.0)
    h0 = jnp.maximum(
        jnp.dot(jnp.concatenate([nx_ref[...], mean], axis=1), w2t_ref[...],
                preferred_element_type=F32) + b2_ref[...], 0.0)
    h0_ref[...] = h0
    y = jnp.dot(h0, w1t_ref[...], preferred_element_type=F32) + b1_ref[...]
    y_ref[...] = jnp.concatenate(
        [y, jnp.ones((ROWS, 1), F32), jnp.zeros((ROWS, FW - GH - 1), F32)],
        axis=1)


def _k4(pp, nx, w2t, b2, w1t, b1):
    full = lambda s: pl.BlockSpec(s, lambda i: (0, 0))
    return pl.pallas_call(
        _k4_body,
        grid=(GRID,),
        in_specs=[
            pl.BlockSpec((2, ROWS, FW), lambda i: (0, i, 0)),
            pl.BlockSpec((ROWS, XE + GH), lambda i: (i, 0)),
            full((XE + 2 * GH, GH)),
            full((1, GH)),
            full((GH, GH)),
            full((1, GH)),
        ],
        out_specs=[
            pl.BlockSpec((ROWS, GH), lambda i: (i, 0)),
            pl.BlockSpec((ROWS, FW), lambda i: (i, 0)),
        ],
        out_shape=[
            jax.ShapeDtypeStruct((N, GH), F32),
            jax.ShapeDtypeStruct((N, FW), F32),
        ],
    )(pp, nx, w2t, b2, w1t, b1)


def _k5_body(pp_ref, h0_ref, w2t_ref, b2_ref, pw1_ref, pb1_ref, pw2_ref,
             pb2_ref, out_ref):
    s = pp_ref[0] + pp_ref[1]
    mean = s[:, :GH] / jnp.maximum(s[:, GH:GH + 1], 1.0)
    h1 = jnp.maximum(
        jnp.dot(jnp.concatenate([h0_ref[...], mean], axis=1), w2t_ref[...],
                preferred_element_type=F32) + b2_ref[...], 0.0)
    r = jnp.maximum(
        jnp.dot(h1, pw1_ref[...], preferred_element_type=F32) + pb1_ref[...],
        0.0)
    out_ref[...] = jnp.maximum(
        jnp.dot(r, pw2_ref[...], preferred_element_type=F32) + pb2_ref[...],
        0.0)


def _k5(pp, h0, w2t, b2, pw1, pb1, pw2, pb2):
    full = lambda s: pl.BlockSpec(s, lambda i: (0, 0))
    return pl.pallas_call(
        _k5_body,
        grid=(GRID,),
        in_specs=[
            pl.BlockSpec((2, ROWS, FW), lambda i: (0, i, 0)),
            pl.BlockSpec((ROWS, GH), lambda i: (i, 0)),
            full((2 * GH, GH)),
            full((1, GH)),
            full((GH, 16)),
            full((1, 16)),
            full((16, PS)),
            full((1, PS)),
        ],
        out_specs=pl.BlockSpec((ROWS, PS), lambda i: (i, 0)),
        out_shape=jax.ShapeDtypeStruct((N, PS), F32),
    )(pp, h0, w2t, b2, pw1, pb1, pw2, pb2)


def kernel(x, u, edge_index, edge_w, loc, params):
    p = params
    xr = x.reshape(N, TW * 8).T.astype(F32)
    locr = loc.reshape(N, 2).T.astype(F32)
    u_i = u.reshape(3).astype(I32)
    wih_t = p['lstm_Wih'].T
    whh_t = p['lstm_Whh'].T
    b2 = (p['lstm_bih'] + p['lstm_bhh']).reshape(1, -1)
    locw_t = p['loc_W'].T
    locb2 = p['loc_b'].reshape(1, -1)

    h, w, gx = _k1(xr, locr, p['w_param'], wih_t, whh_t, b2, locw_t, locb2)

    pr = jnp.arange(256)
    si = jax.nn.one_hot(pr // G, G, dtype=F32)
    sj = jax.nn.one_hot(pr % G, G, dtype=F32)
    dsel = jax.nn.one_hot(jnp.arange(G) * (G + 1), 256, dtype=F32)
    gx2 = _k2(gx, u_i, p['uemb1'], p['uemb2'], p['uemb3'], si, sj, dsel,
              p['einf_W'].T, p['einf_b'].reshape(1, -1),
              p['gg0_W1'].T, p['gg0_b1'].reshape(1, -1),
              p['gg0_W2'].T, p['gg0_b2'].reshape(1, -1),
              p['gg1_W1'].T, p['gg1_b1'].reshape(1, -1),
              p['gg1_W2'].T, p['gg1_b2'].reshape(1, -1))

    nx, y0 = _k3(h, w, gx2, p['gl0_W1'][:, :XE + GH].T,
                 p['gl0_b1'].reshape(1, -1))

    row = edge_index[0, 0].astype(I32)
    col = edge_index[0, 1].astype(I32)
    ew = edge_w.reshape(E).astype(F32)
    npad = EPAD - E
    rowp = jnp.concatenate([row, jnp.zeros((npad,), I32)])
    colp = jnp.concatenate([col, jnp.full((npad,), N, I32)])
    ewp = jnp.concatenate([ew, jnp.zeros((npad,), F32)])
    zrows = jnp.zeros((RPT, FW), F32)
    wl0 = jnp.concatenate([p['gl0_W1'][:, XE + GH], jnp.zeros((16,), F32)])
    wl1 = jnp.concatenate([p['gl1_W1'][:, GH], jnp.zeros((16,), F32)])

    pp0 = _edge_partials(y0, rowp, colp, ewp, wl0, zrows)
    h0, y1 = _k4(pp0[:, :N], nx, p['gl0_W2'].T, p['gl0_b2'].reshape(1, -1),
                 p['gl1_W1'][:, :GH].T, p['gl1_b1'].reshape(1, -1))

    pp1 = _edge_partials(y1, rowp, colp, ewp, wl1, zrows)
    res = _k5(pp1[:, :N], h0, p['gl1_W2'].T, p['gl1_b2'].reshape(1, -1),
              p['pred_W1'].T, p['pred_b1'].reshape(1, -1),
              p['pred_W2'].T, p['pred_b2'].reshape(1, -1))
    return res.reshape(1, N, PS)


# speedup vs baseline: 2.2197x; 2.2197x over previous
import functools

import jax
import jax.numpy as jnp
from jax import lax
from jax.experimental import pallas as pl
from jax.experimental.pallas import tpu as pltpu
from jax.experimental.pallas import tpu_sc as plsc

F32 = jnp.float32
I32 = jnp.int32

TW = 24
N = 10000
XE = 32
LOCE = 12
EH = 16
GH = 64
PS = 6
G = 16
E = 160000
NW = 32
CHUNK = 128
EPW = 5120
EPAD = NW * EPW
NCH = EPW // CHUNK
NPAD = 10240
RPT = NPAD // 16
FW = 80

ROWS = 1000
GRID = 10


def _k1_body(x_ref, loc_ref, wp_ref, wih_ref, whh_ref, b_ref, locw_ref,
             locb_ref, h_ref, w_ref, gx_ref):
    i = pl.program_id(0)
    h = jnp.zeros((ROWS, XE), F32)
    c = jnp.zeros((ROWS, XE), F32)
    wih = wih_ref[...]
    whh = whh_ref[...]
    b = b_ref[...]
    dn = (((0,), (0,)), ((), ()))
    for t in range(TW):
        xt = x_ref[:, t * 8:(t + 1) * 8]
        g = (jnp.dot(xt, wih, preferred_element_type=F32)
             + jnp.dot(h, whh, preferred_element_type=F32) + b)
        ig = jax.nn.sigmoid(g[:, :XE])
        fg = jax.nn.sigmoid(g[:, XE:2 * XE])
        gg = jnp.tanh(g[:, 2 * XE:3 * XE])
        og = jax.nn.sigmoid(g[:, 3 * XE:])
        c = fg * c + ig * gg
        h = og * jnp.tanh(c)
    h_ref[...] = h
    wp = wp_ref[...]
    ex = jnp.exp(wp - jnp.max(wp, axis=1, keepdims=True))
    w = ex / jnp.sum(ex, axis=1, keepdims=True)
    w_ref[...] = w
    loce = (jnp.dot(loc_ref[...], locw_ref[...],
                    preferred_element_type=F32) + locb_ref[...])
    xloc = jnp.concatenate([h, loce], axis=1)
    gxp = lax.dot_general(w, xloc, dn, preferred_element_type=F32)

    @pl.when(i == 0)
    def _():
        gx_ref[...] = jnp.zeros_like(gx_ref)

    gx_ref[...] += gxp


def _k1(xr, locr, wp, wih_t, whh_t, b2, locw_t, locb2):
    full = lambda s: pl.BlockSpec(s, lambda i: (0, 0))
    return pl.pallas_call(
        _k1_body,
        grid=(GRID,),
        in_specs=[
            pl.BlockSpec((ROWS, TW * 8), lambda i: (i, 0)),
            pl.BlockSpec((ROWS, 2), lambda i: (i, 0)),
            pl.BlockSpec((ROWS, G), lambda i: (i, 0)),
            full((8, 4 * XE)),
            full((XE, 4 * XE)),
            full((1, 4 * XE)),
            full((2, LOCE)),
            full((1, LOCE)),
        ],
        out_specs=[
            pl.BlockSpec((ROWS, XE), lambda i: (i, 0)),
            pl.BlockSpec((ROWS, G), lambda i: (i, 0)),
            pl.BlockSpec((G, XE + LOCE), lambda i: (0, 0)),
        ],
        out_shape=[
            jax.ShapeDtypeStruct((N, XE), F32),
            jax.ShapeDtypeStruct((N, G), F32),
            jax.ShapeDtypeStruct((G, XE + LOCE), F32),
        ],
    )(xr, locr, wp, wih_t, whh_t, b2, locw_t, locb2)


def _k2_body(gx_ref, u_ref, ue1_ref, ue2_ref, ue3_ref, si_ref, sj_ref, d_ref,
             ew_t_ref, eb_ref, a1_ref, ab1_ref, a2_ref, ab2_ref, c1_ref,
             cb1_ref, c2_ref, cb2_ref, out_ref):
    gx = gx_ref[...]
    si = si_ref[...]
    sj = sj_ref[...]
    dsel = d_ref[...]
    u0 = u_ref[0]
    u1 = u_ref[1]
    u2 = u_ref[2]
    ue = jnp.concatenate([
        ue1_ref[pl.ds(u0, 1), :],
        ue2_ref[pl.ds(u1, 1), :],
        ue3_ref[pl.ds(u2, 1), :],
    ], axis=1)
    gi = jnp.dot(si, gx, preferred_element_type=F32)
    gj = jnp.dot(sj, gx, preferred_element_type=F32)
    gin = jnp.concatenate(
        [gi, gj, jnp.broadcast_to(ue, (256, 12))], axis=1)
    ge = jnp.maximum(
        jnp.dot(gin, ew_t_ref[...], preferred_element_type=F32)
        + eb_ref[...], 0.0)
    dn = (((0,), (0,)), ((), ()))

    def node_layer(xg, w1t, b1, w2t, b2):
        xi = jnp.dot(si, xg, preferred_element_type=F32)
        m = jnp.maximum(
            jnp.dot(jnp.concatenate([xi, ge], axis=1), w1t,
                    preferred_element_type=F32) + b1, 0.0)
        ssum = lax.dot_general(sj, m, dn, preferred_element_type=F32)
        diag = jnp.dot(dsel, m, preferred_element_type=F32)
        mean = (ssum - diag) * (1.0 / 15.0)
        return jnp.maximum(
            jnp.dot(jnp.concatenate([xg, mean], axis=1), w2t,
                    preferred_element_type=F32) + b2, 0.0)

    g1 = node_layer(gx, a1_ref[...], ab1_ref[...], a2_ref[...], ab2_ref[...])
    g2 = node_layer(g1, c1_ref[...], cb1_ref[...], c2_ref[...], cb2_ref[...])
    out_ref[...] = g2


def _k2(gx, u, ue1, ue2, ue3, si, sj, dsel, ew_t, eb, a1, ab1, a2, ab2, c1,
        cb1, c2, cb2):
    args = (gx, u, ue1, ue2, ue3, si, sj, dsel, ew_t, eb, a1, ab1, a2, ab2,
            c1, cb1, c2, cb2)
    in_specs = [pl.BlockSpec(a.shape, functools.partial(lambda n: (0,) * n,
                                                        a.ndim))
                for a in args]
    in_specs[1] = pl.BlockSpec(memory_space=pltpu.SMEM)
    return pl.pallas_call(
        _k2_body,
        in_specs=in_specs,
        out_specs=pl.BlockSpec((G, GH), lambda: (0, 0)),
        out_shape=jax.ShapeDtypeStruct((G, GH), F32),
    )(*args)


def _k3_body(h_ref, w_ref, gx2_ref, w1t_ref, b1_ref, nx_ref, y_ref):
    h = h_ref[...]
    gnew = jnp.dot(w_ref[...], gx2_ref[...], preferred_element_type=F32)
    nx = jnp.concatenate([h, gnew], axis=1)
    nx_ref[...] = nx
    y = jnp.dot(nx, w1t_ref[...], preferred_element_type=F32) + b1_ref[...]
    y_ref[...] = jnp.concatenate(
        [y, jnp.ones((ROWS, 1), F32), jnp.zeros((ROWS, FW - GH - 1), F32)],
        axis=1)


def _k3(h, w, gx2, w1t, b1):
    full = lambda s: pl.BlockSpec(s, lambda i: (0, 0))
    return pl.pallas_call(
        _k3_body,
        grid=(GRID,),
        in_specs=[
            pl.BlockSpec((ROWS, XE), lambda i: (i, 0)),
            pl.BlockSpec((ROWS, G), lambda i: (i, 0)),
            full((G, GH)),
            full((XE + GH, GH)),
            full((1, GH)),
        ],
        out_specs=[
            pl.BlockSpec((ROWS, XE + GH), lambda i: (i, 0)),
            pl.BlockSpec((ROWS, FW), lambda i: (i, 0)),
        ],
        out_shape=[
            jax.ShapeDtypeStruct((N, XE + GH), F32),
            jax.ShapeDtypeStruct((N, FW), F32),
        ],
    )(h, w, gx2, w1t, b1)


def _sc_body(y_hbm, row_hbm, col_hbm, ew_hbm, wl_hbm, z_hbm, out_hbm,
             row_v, col_v, ew_v, msg_v, wl_v, sums, sem):
    cid = lax.axis_index("c")
    sid = lax.axis_index("s")
    wid = sid * 2 + cid
    pltpu.sync_copy(wl_hbm, wl_v)
    pltpu.sync_copy(z_hbm, sums.at[pl.ds(sid * RPT, RPT)])
    plsc.subcore_barrier()
    wl = [wl_v[pl.ds(j * 16, 16)] for j in range(4)]

    def chunk_body(cn, carry):
        base = wid * EPW + cn * CHUNK
        pltpu.sync_copy(row_hbm.at[pl.ds(base, CHUNK)], row_v)
        pltpu.sync_copy(col_hbm.at[pl.ds(base, CHUNK)], col_v)
        pltpu.sync_copy(ew_hbm.at[pl.ds(base, CHUNK)], ew_v)
        pltpu.async_copy(y_hbm.at[row_v], msg_v, sem).wait()

        def grp_body(g, c2):
            ewg = ew_v[pl.ds(g * 16, 16)]
            for l in range(16):
                ewb = jnp.full((16,), ewg[l], F32)
                e = g * 16 + l
                for j in range(4):
                    v = msg_v[e, pl.ds(j * 16, 16)]
                    msg_v[e, pl.ds(j * 16, 16)] = jnp.maximum(
                        v + ewb * wl[j], 0.0)
            return c2

        lax.fori_loop(0, CHUNK // 16, grp_body, 0)
        pltpu.sync_copy(msg_v, sums.at[col_v], add=True)
        return carry

    lax.fori_loop(0, NCH, chunk_body, 0)
    plsc.subcore_barrier()
    pltpu.sync_copy(sums.at[pl.ds(sid * RPT, RPT)],
                    out_hbm.at[cid, pl.ds(sid * RPT, RPT)])


def _edge_partials(y_ext, rowp, colp, ewp, wl_ext, zrows):
    mesh = plsc.VectorSubcoreMesh(core_axis_name="c", subcore_axis_name="s")
    k = pl.kernel(
        _sc_body,
        mesh=mesh,
        compiler_params=pltpu.CompilerParams(use_tc_tiling_on_sc=False),
        out_type=jax.ShapeDtypeStruct((2, NPAD, FW), F32),
        scratch_types=[
            pltpu.VMEM((CHUNK,), I32),
            pltpu.VMEM((CHUNK,), I32),
            pltpu.VMEM((CHUNK,), F32),
            pltpu.VMEM((CHUNK, FW), F32),
            pltpu.VMEM((FW,), F32),
            pltpu.VMEM_SHARED((NPAD, FW), F32),
            pltpu.SemaphoreType.DMA,
        ],
    )
    return k(y_ext, rowp, colp, ewp, wl_ext, zrows)


def _k4_body(pp_ref, nx_ref, w2t_ref, b2_ref, w1t_ref, b1_ref, h0_ref, y_ref):
    s = pp_ref[0] + pp_ref[1]
    mean = s[:, :GH] / jnp.maximum(s[:, GH:GH + 1], 1.0)
    h0 = jnp.maximum(
        jnp.dot(jnp.concatenate([nx_ref[...], mean], axis=1), w2t_ref[...],
                preferred_element_type=F32) + b2_ref[...], 0.0)
    h0_ref[...] = h0
    y = jnp.dot(h0, w1t_ref[...], preferred_element_type=F32) + b1_ref[...]
    y_ref[...] = jnp.concatenate(
        [y, jnp.ones((ROWS, 1), F32), jnp.zeros((ROWS, FW - GH - 1), F32)],
        axis=1)


def _k4(pp, nx, w2t, b2, w1t, b1):
    full = lambda s: pl.BlockSpec(s, lambda i: (0, 0))
    return pl.pallas_call(
        _k4_body,
        grid=(GRID,),
        in_specs=[
            pl.BlockSpec((2, ROWS, FW), lambda i: (0, i, 0)),
            pl.BlockSpec((ROWS, XE + GH), lambda i: (i, 0)),
            full((XE + 2 * GH, GH)),
            full((1, GH)),
            full((GH, GH)),
            full((1, GH)),
        ],
        out_specs=[
            pl.BlockSpec((ROWS, GH), lambda i: (i, 0)),
            pl.BlockSpec((ROWS, FW), lambda i: (i, 0)),
        ],
        out_shape=[
            jax.ShapeDtypeStruct((N, GH), F32),
            jax.ShapeDtypeStruct((N, FW), F32),
        ],
    )(pp, nx, w2t, b2, w1t, b1)


def _k5_body(pp_ref, h0_ref, w2t_ref, b2_ref, pw1_ref, pb1_ref, pw2_ref,
             pb2_ref, out_ref):
    s = pp_ref[0] + pp_ref[1]
    mean = s[:, :GH] / jnp.maximum(s[:, GH:GH + 1], 1.0)
    h1 = jnp.maximum(
        jnp.dot(jnp.concatenate([h0_ref[...], mean], axis=1), w2t_ref[...],
                preferred_element_type=F32) + b2_ref[...], 0.0)
    r = jnp.maximum(
        jnp.dot(h1, pw1_ref[...], preferred_element_type=F32) + pb1_ref[...],
        0.0)
    out_ref[...] = jnp.maximum(
        jnp.dot(r, pw2_ref[...], preferred_element_type=F32) + pb2_ref[...],
        0.0)


def _k5(pp, h0, w2t, b2, pw1, pb1, pw2, pb2):
    full = lambda s: pl.BlockSpec(s, lambda i: (0, 0))
    return pl.pallas_call(
        _k5_body,
        grid=(GRID,),
        in_specs=[
            pl.BlockSpec((2, ROWS, FW), lambda i: (0, i, 0)),
            pl.BlockSpec((ROWS, GH), lambda i: (i, 0)),
            full((2 * GH, GH)),
            full((1, GH)),
            full((GH, 16)),
            full((1, 16)),
            full((16, PS)),
            full((1, PS)),
        ],
        out_specs=pl.BlockSpec((ROWS, PS), lambda i: (i, 0)),
        out_shape=jax.ShapeDtypeStruct((N, PS), F32),
    )(pp, h0, w2t, b2, pw1, pb1, pw2, pb2)


def kernel(x, u, edge_index, edge_w, loc, params):
    p = params
    xr = x.reshape(N, TW * 8).astype(F32)
    locr = loc.reshape(N, 2).astype(F32)
    u_i = u.reshape(3).astype(I32)
    wih_t = p['lstm_Wih'].T
    whh_t = p['lstm_Whh'].T
    b2 = (p['lstm_bih'] + p['lstm_bhh']).reshape(1, -1)
    locw_t = p['loc_W'].T
    locb2 = p['loc_b'].reshape(1, -1)

    h, w, gx = _k1(xr, locr, p['w_param'], wih_t, whh_t, b2, locw_t, locb2)

    pr = jnp.arange(256)
    si = jax.nn.one_hot(pr // G, G, dtype=F32)
    sj = jax.nn.one_hot(pr % G, G, dtype=F32)
    dsel = jax.nn.one_hot(jnp.arange(G) * (G + 1), 256, dtype=F32)
    gx2 = _k2(gx, u_i, p['uemb1'], p['uemb2'], p['uemb3'], si, sj, dsel,
              p['einf_W'].T, p['einf_b'].reshape(1, -1),
              p['gg0_W1'].T, p['gg0_b1'].reshape(1, -1),
              p['gg0_W2'].T, p['gg0_b2'].reshape(1, -1),
              p['gg1_W1'].T, p['gg1_b1'].reshape(1, -1),
              p['gg1_W2'].T, p['gg1_b2'].reshape(1, -1))

    nx, y0 = _k3(h, w, gx2, p['gl0_W1'][:, :XE + GH].T,
                 p['gl0_b1'].reshape(1, -1))

    row = edge_index[0, 0].astype(I32)
    col = edge_index[0, 1].astype(I32)
    ew = edge_w.reshape(E).astype(F32)
    npad = EPAD - E
    rowp = jnp.concatenate([row, jnp.zeros((npad,), I32)])
    colp = jnp.concatenate([col, jnp.full((npad,), N, I32)])
    ewp = jnp.concatenate([ew, jnp.zeros((npad,), F32)])
    zrows = jnp.zeros((RPT, FW), F32)
    wl0 = jnp.concatenate([p['gl0_W1'][:, XE + GH], jnp.zeros((16,), F32)])
    wl1 = jnp.concatenate([p['gl1_W1'][:, GH], jnp.zeros((16,), F32)])

    pp0 = _edge_partials(y0, rowp, colp, ewp, wl0, zrows)
    h0, y1 = _k4(pp0[:, :N], nx, p['gl0_W2'].T, p['gl0_b2'].reshape(1, -1),
                 p['gl1_W1'][:, :GH].T, p['gl1_b1'].reshape(1, -1))

    pp1 = _edge_partials(y1, rowp, colp, ewp, wl1, zrows)
    res = _k5(pp1[:, :N], h0, p['gl1_W2'].T, p['gl1_b2'].reshape(1, -1),
              p['pred_W1'].T, p['pred_b1'].reshape(1, -1),
              p['pred_W2'].T, p['pred_b2'].reshape(1, -1))
    return res.reshape(1, N, PS)


# speedup vs baseline: 2.9509x; 1.3295x over previous
import functools

import jax
import jax.numpy as jnp
from jax import lax
from jax.experimental import pallas as pl
from jax.experimental.pallas import tpu as pltpu
from jax.experimental.pallas import tpu_sc as plsc

F32 = jnp.float32
I32 = jnp.int32

TW = 24
N = 10000
XE = 32
LOCE = 12
EH = 16
GH = 64
PS = 6
G = 16
E = 160000
NW = 32
CHUNK = 128
EPW = 5120
EPAD = NW * EPW
NCH = EPW // CHUNK
NPAD = 10240
RPT = NPAD // 16
MW = GH
CW = 16

ROWS = 1000
GRID = 10


def _k1_body(x_ref, loc_ref, wp_ref, wih_ref, whh_ref, b_ref, locw_ref,
             locb_ref, h_ref, w_ref, gx_ref):
    i = pl.program_id(0)
    h = jnp.zeros((ROWS, XE), F32)
    c = jnp.zeros((ROWS, XE), F32)
    wih = wih_ref[...]
    whh = whh_ref[...]
    b = b_ref[...]
    dn = (((0,), (0,)), ((), ()))
    for t in range(TW):
        xt = x_ref[:, t * 8:(t + 1) * 8]
        g = (jnp.dot(xt, wih, preferred_element_type=F32)
             + jnp.dot(h, whh, preferred_element_type=F32) + b)
        ig = jax.nn.sigmoid(g[:, :XE])
        fg = jax.nn.sigmoid(g[:, XE:2 * XE])
        gg = jnp.tanh(g[:, 2 * XE:3 * XE])
        og = jax.nn.sigmoid(g[:, 3 * XE:])
        c = fg * c + ig * gg
        h = og * jnp.tanh(c)
    h_ref[...] = h
    wp = wp_ref[...]
    ex = jnp.exp(wp - jnp.max(wp, axis=1, keepdims=True))
    w = ex / jnp.sum(ex, axis=1, keepdims=True)
    w_ref[...] = w
    loce = (jnp.dot(loc_ref[...], locw_ref[...],
                    preferred_element_type=F32) + locb_ref[...])
    xloc = jnp.concatenate([h, loce], axis=1)
    gxp = lax.dot_general(w, xloc, dn, preferred_element_type=F32)

    @pl.when(i == 0)
    def _():
        gx_ref[...] = jnp.zeros_like(gx_ref)

    gx_ref[...] += gxp


def _k1(xr, locr, wp, wih_t, whh_t, b2, locw_t, locb2):
    full = lambda s: pl.BlockSpec(s, lambda i: (0, 0))
    return pl.pallas_call(
        _k1_body,
        grid=(GRID,),
        in_specs=[
            pl.BlockSpec((ROWS, TW * 8), lambda i: (i, 0)),
            pl.BlockSpec((ROWS, 2), lambda i: (i, 0)),
            pl.BlockSpec((ROWS, G), lambda i: (i, 0)),
            full((8, 4 * XE)),
            full((XE, 4 * XE)),
            full((1, 4 * XE)),
            full((2, LOCE)),
            full((1, LOCE)),
        ],
        out_specs=[
            pl.BlockSpec((ROWS, XE), lambda i: (i, 0)),
            pl.BlockSpec((ROWS, G), lambda i: (i, 0)),
            pl.BlockSpec((G, XE + LOCE), lambda i: (0, 0)),
        ],
        out_shape=[
            jax.ShapeDtypeStruct((N, XE), F32),
            jax.ShapeDtypeStruct((N, G), F32),
            jax.ShapeDtypeStruct((G, XE + LOCE), F32),
        ],
    )(xr, locr, wp, wih_t, whh_t, b2, locw_t, locb2)


def _k2_body(gx_ref, u_ref, ue1_ref, ue2_ref, ue3_ref, si_ref, sj_ref, d_ref,
             ew_t_ref, eb_ref, a1_ref, ab1_ref, a2_ref, ab2_ref, c1_ref,
             cb1_ref, c2_ref, cb2_ref, out_ref):
    gx = gx_ref[...]
    si = si_ref[...]
    sj = sj_ref[...]
    dsel = d_ref[...]
    u0 = u_ref[0]
    u1 = u_ref[1]
    u2 = u_ref[2]
    ue = jnp.concatenate([
        ue1_ref[pl.ds(u0, 1), :],
        ue2_ref[pl.ds(u1, 1), :],
        ue3_ref[pl.ds(u2, 1), :],
    ], axis=1)
    gi = jnp.dot(si, gx, preferred_element_type=F32)
    gj = jnp.dot(sj, gx, preferred_element_type=F32)
    gin = jnp.concatenate(
        [gi, gj, jnp.broadcast_to(ue, (256, 12))], axis=1)
    ge = jnp.maximum(
        jnp.dot(gin, ew_t_ref[...], preferred_element_type=F32)
        + eb_ref[...], 0.0)
    dn = (((0,), (0,)), ((), ()))

    def node_layer(xg, w1t, b1, w2t, b2):
        xi = jnp.dot(si, xg, preferred_element_type=F32)
        m = jnp.maximum(
            jnp.dot(jnp.concatenate([xi, ge], axis=1), w1t,
                    preferred_element_type=F32) + b1, 0.0)
        ssum = lax.dot_general(sj, m, dn, preferred_element_type=F32)
        diag = jnp.dot(dsel, m, preferred_element_type=F32)
        mean = (ssum - diag) * (1.0 / 15.0)
        return jnp.maximum(
            jnp.dot(jnp.concatenate([xg, mean], axis=1), w2t,
                    preferred_element_type=F32) + b2, 0.0)

    g1 = node_layer(gx, a1_ref[...], ab1_ref[...], a2_ref[...], ab2_ref[...])
    g2 = node_layer(g1, c1_ref[...], cb1_ref[...], c2_ref[...], cb2_ref[...])
    out_ref[...] = g2


def _k2(gx, u, ue1, ue2, ue3, si, sj, dsel, ew_t, eb, a1, ab1, a2, ab2, c1,
        cb1, c2, cb2):
    args = (gx, u, ue1, ue2, ue3, si, sj, dsel, ew_t, eb, a1, ab1, a2, ab2,
            c1, cb1, c2, cb2)
    in_specs = [pl.BlockSpec(a.shape, functools.partial(lambda n: (0,) * n,
                                                        a.ndim))
                for a in args]
    in_specs[1] = pl.BlockSpec(memory_space=pltpu.SMEM)
    return pl.pallas_call(
        _k2_body,
        in_specs=in_specs,
        out_specs=pl.BlockSpec((G, GH), lambda: (0, 0)),
        out_shape=jax.ShapeDtypeStruct((G, GH), F32),
    )(*args)


def _k3_body(h_ref, w_ref, gx2_ref, w1t_ref, b1_ref, nx_ref, y_ref):
    h = h_ref[...]
    gnew = jnp.dot(w_ref[...], gx2_ref[...], preferred_element_type=F32)
    nx = jnp.concatenate([h, gnew], axis=1)
    nx_ref[...] = nx
    y_ref[...] = (jnp.dot(nx, w1t_ref[...], preferred_element_type=F32)
                  + b1_ref[...])


def _k3(h, w, gx2, w1t, b1):
    full = lambda s: pl.BlockSpec(s, lambda i: (0, 0))
    return pl.pallas_call(
        _k3_body,
        grid=(GRID,),
        in_specs=[
            pl.BlockSpec((ROWS, XE), lambda i: (i, 0)),
            pl.BlockSpec((ROWS, G), lambda i: (i, 0)),
            full((G, GH)),
            full((XE + GH, GH)),
            full((1, GH)),
        ],
        out_specs=[
            pl.BlockSpec((ROWS, XE + GH), lambda i: (i, 0)),
            pl.BlockSpec((ROWS, MW), lambda i: (i, 0)),
        ],
        out_shape=[
            jax.ShapeDtypeStruct((N, XE + GH), F32),
            jax.ShapeDtypeStruct((N, MW), F32),
        ],
    )(h, w, gx2, w1t, b1)


def _make_sc_kernel(do_counts):
    def body(*refs):
        if do_counts:
            (y_hbm, row_hbm, col_hbm, ew_hbm, wl_hbm, z_hbm, zc_hbm, ones_hbm,
             out_hbm, cnt_hbm, row0, row1, col0, col1, ewall, msg0, msg1,
             wl_v, ones_v, sums, csum, sem0, sem1) = refs
        else:
            (y_hbm, row_hbm, col_hbm, ew_hbm, wl_hbm, z_hbm,
             out_hbm, row0, row1, col0, col1, ewall, msg0, msg1,
             wl_v, sums, sem0, sem1) = refs
        cid = lax.axis_index("c")
        sid = lax.axis_index("s")
        wid = sid * 2 + cid
        base = wid * EPW
        rowb = [row0, row1]
        colb = [col0, col1]
        msgb = [msg0, msg1]
        semb = [sem0, sem1]
        pltpu.sync_copy(ew_hbm.at[pl.ds(base, EPW)], ewall)
        pltpu.sync_copy(wl_hbm, wl_v)
        pltpu.sync_copy(z_hbm, sums.at[pl.ds(sid * RPT, RPT)])
        if do_counts:
            pltpu.sync_copy(ones_hbm, ones_v)
            pltpu.sync_copy(zc_hbm, csum.at[pl.ds(sid * RPT, RPT)])
        plsc.subcore_barrier()
        wl = [wl_v[pl.ds(j * 16, 16)] for j in range(4)]

        pltpu.sync_copy(row_hbm.at[pl.ds(base, CHUNK)], rowb[0])
        pltpu.sync_copy(col_hbm.at[pl.ds(base, CHUNK)], colb[0])
        pltpu.async_copy(y_hbm.at[rowb[0]], msgb[0], semb[0])

        def compute_scatter(cn, b):
            def grp_body(g, c2):
                ewg = ewall[pl.ds(cn * CHUNK + g * 16, 16)]
                for l in range(16):
                    ewb = jnp.full((16,), ewg[l], F32)
                    e = g * 16 + l
                    for j in range(4):
                        v = msgb[b][e, pl.ds(j * 16, 16)]
                        msgb[b][e, pl.ds(j * 16, 16)] = jnp.maximum(
                            v + ewb * wl[j], 0.0)
                return c2

            lax.fori_loop(0, CHUNK // 16, grp_body, 0)
            pltpu.sync_copy(msgb[b], sums.at[colb[b]], add=True)
            if do_counts:
                pltpu.sync_copy(ones_v, csum.at[colb[b]], add=True)

        def outer(k, carry):
            for b in range(2):
                cn = k * 2 + b
                nb = 1 - b
                nxt = base + jnp.minimum(cn + 1, NCH - 1) * CHUNK
                pltpu.sync_copy(row_hbm.at[pl.ds(nxt, CHUNK)], rowb[nb])
                pltpu.sync_copy(col_hbm.at[pl.ds(nxt, CHUNK)], colb[nb])
                pltpu.async_copy(y_hbm.at[rowb[nb]], msgb[nb], semb[nb])
                pltpu.make_async_copy(y_hbm.at[rowb[b]], msgb[b],
                                      semb[b]).wait()
                compute_scatter(cn, b)
            return carry

        lax.fori_loop(0, NCH // 2, outer, 0)
        pltpu.make_async_copy(y_hbm.at[rowb[0]], msgb[0], semb[0]).wait()
        plsc.subcore_barrier()
        pltpu.sync_copy(sums.at[pl.ds(sid * RPT, RPT)],
                        out_hbm.at[cid, pl.ds(sid * RPT, RPT)])
        if do_counts:
            pltpu.sync_copy(csum.at[pl.ds(sid * RPT, RPT)],
                            cnt_hbm.at[cid, pl.ds(sid * RPT, RPT)])

    out_type = [jax.ShapeDtypeStruct((2, NPAD, MW), F32)]
    scratch = [
        pltpu.VMEM((CHUNK,), I32),
        pltpu.VMEM((CHUNK,), I32),
        pltpu.VMEM((CHUNK,), I32),
        pltpu.VMEM((CHUNK,), I32),
        pltpu.VMEM((EPW,), F32),
        pltpu.VMEM((CHUNK, MW), F32),
        pltpu.VMEM((CHUNK, MW), F32),
        pltpu.VMEM((MW,), F32),
    ]
    if do_counts:
        out_type.append(jax.ShapeDtypeStruct((2, NPAD, CW), F32))
        scratch.append(pltpu.VMEM((CHUNK, CW), F32))
    scratch.append(pltpu.VMEM_SHARED((NPAD, MW), F32))
    if do_counts:
        scratch.append(pltpu.VMEM_SHARED((NPAD, CW), F32))
    scratch += [pltpu.SemaphoreType.DMA, pltpu.SemaphoreType.DMA]
    mesh = plsc.VectorSubcoreMesh(core_axis_name="c", subcore_axis_name="s")
    return pl.kernel(
        body,
        mesh=mesh,
        compiler_params=pltpu.CompilerParams(use_tc_tiling_on_sc=False),
        out_type=out_type if do_counts else out_type[0],
        scratch_types=scratch,
    )


_make_sc_kernel = functools.cache(_make_sc_kernel)


def _sc_pass0(*args):
    return _make_sc_kernel(True)(*args)


def _sc_pass1(*args):
    return _make_sc_kernel(False)(*args)


def _k4_body(pp_ref, pc_ref, nx_ref, w2t_ref, b2_ref, w1t_ref, b1_ref,
             h0_ref, y_ref):
    s = pp_ref[0] + pp_ref[1]
    cnt = pc_ref[0, :, :1] + pc_ref[1, :, :1]
    mean = s / jnp.maximum(cnt, 1.0)
    h0 = jnp.maximum(
        jnp.dot(jnp.concatenate([nx_ref[...], mean], axis=1), w2t_ref[...],
                preferred_element_type=F32) + b2_ref[...], 0.0)
    h0_ref[...] = h0
    y_ref[...] = (jnp.dot(h0, w1t_ref[...], preferred_element_type=F32)
                  + b1_ref[...])


def _k4(pp, pc, nx, w2t, b2, w1t, b1):
    full = lambda s: pl.BlockSpec(s, lambda i: (0, 0))
    return pl.pallas_call(
        _k4_body,
        grid=(GRID,),
        in_specs=[
            pl.BlockSpec((2, ROWS, MW), lambda i: (0, i, 0)),
            pl.BlockSpec((2, ROWS, CW), lambda i: (0, i, 0)),
            pl.BlockSpec((ROWS, XE + GH), lambda i: (i, 0)),
            full((XE + 2 * GH, GH)),
            full((1, GH)),
            full((GH, GH)),
            full((1, GH)),
        ],
        out_specs=[
            pl.BlockSpec((ROWS, GH), lambda i: (i, 0)),
            pl.BlockSpec((ROWS, MW), lambda i: (i, 0)),
        ],
        out_shape=[
            jax.ShapeDtypeStruct((N, GH), F32),
            jax.ShapeDtypeStruct((N, MW), F32),
        ],
    )(pp, pc, nx, w2t, b2, w1t, b1)


def _k5_body(pp_ref, pc_ref, h0_ref, w2t_ref, b2_ref, pw1_ref, pb1_ref,
             pw2_ref, pb2_ref, out_ref):
    s = pp_ref[0] + pp_ref[1]
    cnt = pc_ref[0, :, :1] + pc_ref[1, :, :1]
    mean = s / jnp.maximum(cnt, 1.0)
    h1 = jnp.maximum(
        jnp.dot(jnp.concatenate([h0_ref[...], mean], axis=1), w2t_ref[...],
                preferred_element_type=F32) + b2_ref[...], 0.0)
    r = jnp.maximum(
        jnp.dot(h1, pw1_ref[...], preferred_element_type=F32) + pb1_ref[...],
        0.0)
    out_ref[...] = jnp.maximum(
        jnp.dot(r, pw2_ref[...], preferred_element_type=F32) + pb2_ref[...],
        0.0)


def _k5(pp, pc, h0, w2t, b2, pw1, pb1, pw2, pb2):
    full = lambda s: pl.BlockSpec(s, lambda i: (0, 0))
    return pl.pallas_call(
        _k5_body,
        grid=(GRID,),
        in_specs=[
            pl.BlockSpec((2, ROWS, MW), lambda i: (0, i, 0)),
            pl.BlockSpec((2, ROWS, CW), lambda i: (0, i, 0)),
            pl.BlockSpec((ROWS, GH), lambda i: (i, 0)),
            full((2 * GH, GH)),
            full((1, GH)),
            full((GH, 16)),
            full((1, 16)),
            full((16, PS)),
            full((1, PS)),
        ],
        out_specs=pl.BlockSpec((ROWS, PS), lambda i: (i, 0)),
        out_shape=jax.ShapeDtypeStruct((N, PS), F32),
    )(pp, pc, h0, w2t, b2, pw1, pb1, pw2, pb2)


def kernel(x, u, edge_index, edge_w, loc, params):
    p = params
    xr = x.reshape(N, TW * 8).astype(F32)
    locr = loc.reshape(N, 2).astype(F32)
    u_i = u.reshape(3).astype(I32)
    wih_t = p['lstm_Wih'].T
    whh_t = p['lstm_Whh'].T
    b2 = (p['lstm_bih'] + p['lstm_bhh']).reshape(1, -1)
    locw_t = p['loc_W'].T
    locb2 = p['loc_b'].reshape(1, -1)

    h, w, gx = _k1(xr, locr, p['w_param'], wih_t, whh_t, b2, locw_t, locb2)

    pr = jnp.arange(256)
    si = jax.nn.one_hot(pr // G, G, dtype=F32)
    sj = jax.nn.one_hot(pr % G, G, dtype=F32)
    dsel = jax.nn.one_hot(jnp.arange(G) * (G + 1), 256, dtype=F32)
    gx2 = _k2(gx, u_i, p['uemb1'], p['uemb2'], p['uemb3'], si, sj, dsel,
              p['einf_W'].T, p['einf_b'].reshape(1, -1),
              p['gg0_W1'].T, p['gg0_b1'].reshape(1, -1),
              p['gg0_W2'].T, p['gg0_b2'].reshape(1, -1),
              p['gg1_W1'].T, p['gg1_b1'].reshape(1, -1),
              p['gg1_W2'].T, p['gg1_b2'].reshape(1, -1))

    nx, y0 = _k3(h, w, gx2, p['gl0_W1'][:, :XE + GH].T,
                 p['gl0_b1'].reshape(1, -1))

    row = edge_index[0, 0].astype(I32)
    col = edge_index[0, 1].astype(I32)
    ew = edge_w.reshape(E).astype(F32)
    npad = EPAD - E
    rowp = jnp.concatenate([row, jnp.zeros((npad,), I32)])
    colp = jnp.concatenate([col, jnp.full((npad,), N, I32)])
    ewp = jnp.concatenate([ew, jnp.zeros((npad,), F32)])
    zrows = jnp.zeros((RPT, MW), F32)
    zc = jnp.zeros((RPT, CW), F32)
    onesb = jnp.ones((CHUNK, CW), F32)
    wl0 = p['gl0_W1'][:, XE + GH]
    wl1 = p['gl1_W1'][:, GH]

    pp0, pc = _sc_pass0(y0, rowp, colp, ewp, wl0, zrows, zc, onesb)
    pcn = pc[:, :N]
    h0, y1 = _k4(pp0[:, :N], pcn, nx,
                 p['gl0_W2'].T, p['gl0_b2'].reshape(1, -1),
                 p['gl1_W1'][:, :GH].T, p['gl1_b1'].reshape(1, -1))

    pp1 = _sc_pass1(y1, rowp, colp, ewp, wl1, zrows)
    res = _k5(pp1[:, :N], pcn, h0, p['gl1_W2'].T, p['gl1_b2'].reshape(1, -1),
              p['pred_W1'].T, p['pred_b1'].reshape(1, -1),
              p['pred_W2'].T, p['pred_b2'].reshape(1, -1))
    return res.reshape(1, N, PS)


# speedup vs baseline: 3.0139x; 1.0213x over previous
import functools

import jax
import jax.numpy as jnp
from jax import lax
from jax.experimental import pallas as pl
from jax.experimental.pallas import tpu as pltpu
from jax.experimental.pallas import tpu_sc as plsc

F32 = jnp.float32
I32 = jnp.int32

TW = 24
N = 10000
XE = 32
LOCE = 12
EH = 16
GH = 64
PS = 6
G = 16
E = 160000
NW = 32
CHUNK = 128
EPW = 5120
EPAD = NW * EPW
NCH = EPW // CHUNK
NPAD = 10240
RPT = NPAD // 16
MW = GH
CW = 16

ROWS = 1000
GRID = 10


def _k1_body(x_ref, loc_ref, wp_ref, wih_ref, whh_ref, b_ref, locw_ref,
             locb_ref, h_ref, w_ref, gx_ref):
    i = pl.program_id(0)
    h = jnp.zeros((ROWS, XE), F32)
    c = jnp.zeros((ROWS, XE), F32)
    wih = wih_ref[...]
    whh = whh_ref[...]
    b = b_ref[...]
    dn = (((0,), (0,)), ((), ()))
    for t in range(TW):
        xt = x_ref[:, t * 8:(t + 1) * 8]
        g = (jnp.dot(xt, wih, preferred_element_type=F32)
             + jnp.dot(h, whh, preferred_element_type=F32) + b)
        ig = jax.nn.sigmoid(g[:, :XE])
        fg = jax.nn.sigmoid(g[:, XE:2 * XE])
        gg = jnp.tanh(g[:, 2 * XE:3 * XE])
        og = jax.nn.sigmoid(g[:, 3 * XE:])
        c = fg * c + ig * gg
        h = og * jnp.tanh(c)
    h_ref[...] = h
    wp = wp_ref[...]
    ex = jnp.exp(wp - jnp.max(wp, axis=1, keepdims=True))
    w = ex / jnp.sum(ex, axis=1, keepdims=True)
    w_ref[...] = w
    loce = (jnp.dot(loc_ref[...], locw_ref[...],
                    preferred_element_type=F32) + locb_ref[...])
    xloc = jnp.concatenate([h, loce], axis=1)
    gxp = lax.dot_general(w, xloc, dn, preferred_element_type=F32)

    @pl.when(i == 0)
    def _():
        gx_ref[...] = jnp.zeros_like(gx_ref)

    gx_ref[...] += gxp


def _k1(xr, locr, wp, wih_t, whh_t, b2, locw_t, locb2):
    full = lambda s: pl.BlockSpec(s, lambda i: (0, 0))
    return pl.pallas_call(
        _k1_body,
        grid=(GRID,),
        in_specs=[
            pl.BlockSpec((ROWS, TW * 8), lambda i: (i, 0)),
            pl.BlockSpec((ROWS, 2), lambda i: (i, 0)),
            pl.BlockSpec((ROWS, G), lambda i: (i, 0)),
            full((8, 4 * XE)),
            full((XE, 4 * XE)),
            full((1, 4 * XE)),
            full((2, LOCE)),
            full((1, LOCE)),
        ],
        out_specs=[
            pl.BlockSpec((ROWS, XE), lambda i: (i, 0)),
            pl.BlockSpec((ROWS, G), lambda i: (i, 0)),
            pl.BlockSpec((G, XE + LOCE), lambda i: (0, 0)),
        ],
        out_shape=[
            jax.ShapeDtypeStruct((N, XE), F32),
            jax.ShapeDtypeStruct((N, G), F32),
            jax.ShapeDtypeStruct((G, XE + LOCE), F32),
        ],
    )(xr, locr, wp, wih_t, whh_t, b2, locw_t, locb2)


def _k2_body(gx_ref, u_ref, ue1_ref, ue2_ref, ue3_ref, si_ref, sj_ref, d_ref,
             ew_t_ref, eb_ref, a1_ref, ab1_ref, a2_ref, ab2_ref, c1_ref,
             cb1_ref, c2_ref, cb2_ref, out_ref):
    gx = gx_ref[...]
    si = si_ref[...]
    sj = sj_ref[...]
    dsel = d_ref[...]
    u0 = u_ref[0]
    u1 = u_ref[1]
    u2 = u_ref[2]
    ue = jnp.concatenate([
        ue1_ref[pl.ds(u0, 1), :],
        ue2_ref[pl.ds(u1, 1), :],
        ue3_ref[pl.ds(u2, 1), :],
    ], axis=1)
    gi = jnp.dot(si, gx, preferred_element_type=F32)
    gj = jnp.dot(sj, gx, preferred_element_type=F32)
    gin = jnp.concatenate(
        [gi, gj, jnp.broadcast_to(ue, (256, 12))], axis=1)
    ge = jnp.maximum(
        jnp.dot(gin, ew_t_ref[...], preferred_element_type=F32)
        + eb_ref[...], 0.0)
    dn = (((0,), (0,)), ((), ()))

    def node_layer(xg, w1t, b1, w2t, b2):
        xi = jnp.dot(si, xg, preferred_element_type=F32)
        m = jnp.maximum(
            jnp.dot(jnp.concatenate([xi, ge], axis=1), w1t,
                    preferred_element_type=F32) + b1, 0.0)
        ssum = lax.dot_general(sj, m, dn, preferred_element_type=F32)
        diag = jnp.dot(dsel, m, preferred_element_type=F32)
        mean = (ssum - diag) * (1.0 / 15.0)
        return jnp.maximum(
            jnp.dot(jnp.concatenate([xg, mean], axis=1), w2t,
                    preferred_element_type=F32) + b2, 0.0)

    g1 = node_layer(gx, a1_ref[...], ab1_ref[...], a2_ref[...], ab2_ref[...])
    g2 = node_layer(g1, c1_ref[...], cb1_ref[...], c2_ref[...], cb2_ref[...])
    out_ref[...] = g2


def _k2(gx, u, ue1, ue2, ue3, si, sj, dsel, ew_t, eb, a1, ab1, a2, ab2, c1,
        cb1, c2, cb2):
    args = (gx, u, ue1, ue2, ue3, si, sj, dsel, ew_t, eb, a1, ab1, a2, ab2,
            c1, cb1, c2, cb2)
    in_specs = [pl.BlockSpec(a.shape, functools.partial(lambda n: (0,) * n,
                                                        a.ndim))
                for a in args]
    in_specs[1] = pl.BlockSpec(memory_space=pltpu.SMEM)
    return pl.pallas_call(
        _k2_body,
        in_specs=in_specs,
        out_specs=pl.BlockSpec((G, GH), lambda: (0, 0)),
        out_shape=jax.ShapeDtypeStruct((G, GH), F32),
    )(*args)


def _k3_body(h_ref, w_ref, gx2_ref, w1t_ref, b1_ref, nx_ref, y_ref):
    h = h_ref[...]
    gnew = jnp.dot(w_ref[...], gx2_ref[...], preferred_element_type=F32)
    nx = jnp.concatenate([h, gnew], axis=1)
    nx_ref[...] = nx
    y_ref[...] = (jnp.dot(nx, w1t_ref[...], preferred_element_type=F32)
                  + b1_ref[...])


def _k3(h, w, gx2, w1t, b1):
    full = lambda s: pl.BlockSpec(s, lambda i: (0, 0))
    return pl.pallas_call(
        _k3_body,
        grid=(GRID,),
        in_specs=[
            pl.BlockSpec((ROWS, XE), lambda i: (i, 0)),
            pl.BlockSpec((ROWS, G), lambda i: (i, 0)),
            full((G, GH)),
            full((XE + GH, GH)),
            full((1, GH)),
        ],
        out_specs=[
            pl.BlockSpec((ROWS, XE + GH), lambda i: (i, 0)),
            pl.BlockSpec((ROWS, MW), lambda i: (i, 0)),
        ],
        out_shape=[
            jax.ShapeDtypeStruct((N, XE + GH), F32),
            jax.ShapeDtypeStruct((N, MW), F32),
        ],
    )(h, w, gx2, w1t, b1)


NBUF = 4


def _make_sc_kernel(do_counts):
    def body(*refs):
        if do_counts:
            (y_hbm, row_hbm, col_hbm, ew_hbm, wl_hbm, z_hbm, zc_hbm, ones_hbm,
             out_hbm, cnt_hbm,
             r0, r1, r2, r3, c0, c1, c2, c3, m0, m1, m2, m3,
             ewall, wl_v, ones_v, sums, csum,
             ir0, ir1, ir2, ir3, ic0, ic1, ic2, ic3,
             g0, g1, g2, g3, s0, s1, s2, s3) = refs
        else:
            (y_hbm, row_hbm, col_hbm, ew_hbm, wl_hbm, z_hbm,
             out_hbm,
             r0, r1, r2, r3, c0, c1, c2, c3, m0, m1, m2, m3,
             ewall, wl_v, sums,
             ir0, ir1, ir2, ir3, ic0, ic1, ic2, ic3,
             g0, g1, g2, g3, s0, s1, s2, s3) = refs
        cid = lax.axis_index("c")
        sid = lax.axis_index("s")
        wid = sid * 2 + cid
        base = wid * EPW
        rowb = [r0, r1, r2, r3]
        colb = [c0, c1, c2, c3]
        msgb = [m0, m1, m2, m3]
        irsem = [ir0, ir1, ir2, ir3]
        icsem = [ic0, ic1, ic2, ic3]
        gsem = [g0, g1, g2, g3]
        ssem = [s0, s1, s2, s3]
        pltpu.sync_copy(ew_hbm.at[pl.ds(base, EPW)], ewall)
        pltpu.sync_copy(wl_hbm, wl_v)
        pltpu.sync_copy(z_hbm, sums.at[pl.ds(sid * RPT, RPT)])
        if do_counts:
            pltpu.sync_copy(ones_hbm, ones_v)
            pltpu.sync_copy(zc_hbm, csum.at[pl.ds(sid * RPT, RPT)])
        plsc.subcore_barrier()
        wl = [wl_v[pl.ds(j * 16, 16)] for j in range(4)]

        def idx_copy(chunk, slot):
            off = base + chunk * CHUNK
            pltpu.async_copy(row_hbm.at[pl.ds(off, CHUNK)], rowb[slot],
                             irsem[slot])
            pltpu.async_copy(col_hbm.at[pl.ds(off, CHUNK)], colb[slot],
                             icsem[slot])

        def idx_wait(slot):
            pltpu.make_async_copy(row_hbm.at[pl.ds(0, CHUNK)], rowb[slot],
                                  irsem[slot]).wait()
            pltpu.make_async_copy(col_hbm.at[pl.ds(0, CHUNK)], colb[slot],
                                  icsem[slot]).wait()

        def gather(slot):
            pltpu.async_copy(y_hbm.at[rowb[slot]], msgb[slot], gsem[slot])

        def gather_wait(slot):
            pltpu.make_async_copy(y_hbm.at[rowb[slot]], msgb[slot],
                                  gsem[slot]).wait()

        def scatter(slot):
            pltpu.async_copy(msgb[slot], sums.at[colb[slot]], ssem[slot],
                             add=True)

        def scatter_wait(slot):
            pltpu.make_async_copy(msgb[slot], sums.at[colb[slot]],
                                  ssem[slot]).wait()

        def compute(cn, b):
            def grp_body(g, c2):
                ewg = ewall[pl.ds(cn * CHUNK + g * 16, 16)]
                for l in range(16):
                    ewb = jnp.full((16,), ewg[l], F32)
                    e = g * 16 + l
                    for j in range(4):
                        v = msgb[b][e, pl.ds(j * 16, 16)]
                        msgb[b][e, pl.ds(j * 16, 16)] = jnp.maximum(
                            v + ewb * wl[j], 0.0)
                return c2

            lax.fori_loop(0, CHUNK // 16, grp_body, 0)

        idx_copy(0, 0)
        idx_copy(1, 1)
        idx_wait(0)
        gather(0)

        def outer(k, carry):
            for b in range(NBUF):
                cn = k * NBUF + b
                f = (b + 2) % NBUF
                nb = (b + 1) % NBUF
                if b >= 2:
                    scatter_wait(f)
                else:
                    @pl.when(k >= 1)
                    def _():
                        scatter_wait(f)
                idx_copy(jnp.minimum(cn + 2, NCH - 1), f)
                idx_wait(nb)
                gather(nb)
                gather_wait(b)
                compute(cn, b)
                scatter(b)
                if do_counts:
                    pltpu.sync_copy(ones_v, csum.at[colb[b]], add=True)
            return carry

        lax.fori_loop(0, NCH // NBUF, outer, 0)
        gather_wait(0)
        idx_wait(1)
        scatter_wait(2)
        scatter_wait(3)
        plsc.subcore_barrier()
        pltpu.sync_copy(sums.at[pl.ds(sid * RPT, RPT)],
                        out_hbm.at[cid, pl.ds(sid * RPT, RPT)])
        if do_counts:
            pltpu.sync_copy(csum.at[pl.ds(sid * RPT, RPT)],
                            cnt_hbm.at[cid, pl.ds(sid * RPT, RPT)])

    out_type = [jax.ShapeDtypeStruct((2, NPAD, MW), F32)]
    scratch = (
        [pltpu.VMEM((CHUNK,), I32)] * 8
        + [pltpu.VMEM((CHUNK, MW), F32)] * 4
        + [pltpu.VMEM((EPW,), F32), pltpu.VMEM((MW,), F32)]
    )
    if do_counts:
        out_type.append(jax.ShapeDtypeStruct((2, NPAD, CW), F32))
        scratch.append(pltpu.VMEM((CHUNK, CW), F32))
    scratch.append(pltpu.VMEM_SHARED((NPAD, MW), F32))
    if do_counts:
        scratch.append(pltpu.VMEM_SHARED((NPAD, CW), F32))
    scratch += [pltpu.SemaphoreType.DMA] * 16
    mesh = plsc.VectorSubcoreMesh(core_axis_name="c", subcore_axis_name="s")
    return pl.kernel(
        body,
        mesh=mesh,
        compiler_params=pltpu.CompilerParams(use_tc_tiling_on_sc=False),
        out_type=out_type if do_counts else out_type[0],
        scratch_types=scratch,
    )


_make_sc_kernel = functools.cache(_make_sc_kernel)


def _sc_pass0(*args):
    return _make_sc_kernel(True)(*args)


def _sc_pass1(*args):
    return _make_sc_kernel(False)(*args)


def _k4_body(pp_ref, pc_ref, nx_ref, w2t_ref, b2_ref, w1t_ref, b1_ref,
             h0_ref, y_ref):
    s = pp_ref[0] + pp_ref[1]
    cnt = pc_ref[0, :, :1] + pc_ref[1, :, :1]
    mean = s / jnp.maximum(cnt, 1.0)
    h0 = jnp.maximum(
        jnp.dot(jnp.concatenate([nx_ref[...], mean], axis=1), w2t_ref[...],
                preferred_element_type=F32) + b2_ref[...], 0.0)
    h0_ref[...] = h0
    y_ref[...] = (jnp.dot(h0, w1t_ref[...], preferred_element_type=F32)
                  + b1_ref[...])


def _k4(pp, pc, nx, w2t, b2, w1t, b1):
    full = lambda s: pl.BlockSpec(s, lambda i: (0, 0))
    return pl.pallas_call(
        _k4_body,
        grid=(GRID,),
        in_specs=[
            pl.BlockSpec((2, ROWS, MW), lambda i: (0, i, 0)),
            pl.BlockSpec((2, ROWS, CW), lambda i: (0, i, 0)),
            pl.BlockSpec((ROWS, XE + GH), lambda i: (i, 0)),
            full((XE + 2 * GH, GH)),
            full((1, GH)),
            full((GH, GH)),
            full((1, GH)),
        ],
        out_specs=[
            pl.BlockSpec((ROWS, GH), lambda i: (i, 0)),
            pl.BlockSpec((ROWS, MW), lambda i: (i, 0)),
        ],
        out_shape=[
            jax.ShapeDtypeStruct((N, GH), F32),
            jax.ShapeDtypeStruct((N, MW), F32),
        ],
    )(pp, pc, nx, w2t, b2, w1t, b1)


def _k5_body(pp_ref, pc_ref, h0_ref, w2t_ref, b2_ref, pw1_ref, pb1_ref,
             pw2_ref, pb2_ref, out_ref):
    s = pp_ref[0] + pp_ref[1]
    cnt = pc_ref[0, :, :1] + pc_ref[1, :, :1]
    mean = s / jnp.maximum(cnt, 1.0)
    h1 = jnp.maximum(
        jnp.dot(jnp.concatenate([h0_ref[...], mean], axis=1), w2t_ref[...],
                preferred_element_type=F32) + b2_ref[...], 0.0)
    r = jnp.maximum(
        jnp.dot(h1, pw1_ref[...], preferred_element_type=F32) + pb1_ref[...],
        0.0)
    out_ref[...] = jnp.maximum(
        jnp.dot(r, pw2_ref[...], preferred_element_type=F32) + pb2_ref[...],
        0.0)


def _k5(pp, pc, h0, w2t, b2, pw1, pb1, pw2, pb2):
    full = lambda s: pl.BlockSpec(s, lambda i: (0, 0))
    return pl.pallas_call(
        _k5_body,
        grid=(GRID,),
        in_specs=[
            pl.BlockSpec((2, ROWS, MW), lambda i: (0, i, 0)),
            pl.BlockSpec((2, ROWS, CW), lambda i: (0, i, 0)),
            pl.BlockSpec((ROWS, GH), lambda i: (i, 0)),
            full((2 * GH, GH)),
            full((1, GH)),
            full((GH, 16)),
            full((1, 16)),
            full((16, PS)),
            full((1, PS)),
        ],
        out_specs=pl.BlockSpec((ROWS, PS), lambda i: (i, 0)),
        out_shape=jax.ShapeDtypeStruct((N, PS), F32),
    )(pp, pc, h0, w2t, b2, pw1, pb1, pw2, pb2)


def kernel(x, u, edge_index, edge_w, loc, params):
    p = params
    xr = x.reshape(N, TW * 8).astype(F32)
    locr = loc.reshape(N, 2).astype(F32)
    u_i = u.reshape(3).astype(I32)
    wih_t = p['lstm_Wih'].T
    whh_t = p['lstm_Whh'].T
    b2 = (p['lstm_bih'] + p['lstm_bhh']).reshape(1, -1)
    locw_t = p['loc_W'].T
    locb2 = p['loc_b'].reshape(1, -1)

    h, w, gx = _k1(xr, locr, p['w_param'], wih_t, whh_t, b2, locw_t, locb2)

    pr = jnp.arange(256)
    si = jax.nn.one_hot(pr // G, G, dtype=F32)
    sj = jax.nn.one_hot(pr % G, G, dtype=F32)
    dsel = jax.nn.one_hot(jnp.arange(G) * (G + 1), 256, dtype=F32)
    gx2 = _k2(gx, u_i, p['uemb1'], p['uemb2'], p['uemb3'], si, sj, dsel,
              p['einf_W'].T, p['einf_b'].reshape(1, -1),
              p['gg0_W1'].T, p['gg0_b1'].reshape(1, -1),
              p['gg0_W2'].T, p['gg0_b2'].reshape(1, -1),
              p['gg1_W1'].T, p['gg1_b1'].reshape(1, -1),
              p['gg1_W2'].T, p['gg1_b2'].reshape(1, -1))

    nx, y0 = _k3(h, w, gx2, p['gl0_W1'][:, :XE + GH].T,
                 p['gl0_b1'].reshape(1, -1))

    row = edge_index[0, 0].astype(I32)
    col = edge_index[0, 1].astype(I32)
    ew = edge_w.reshape(E).astype(F32)
    npad = EPAD - E
    rowp = jnp.concatenate([row, jnp.zeros((npad,), I32)])
    colp = jnp.concatenate([col, jnp.full((npad,), N, I32)])
    ewp = jnp.concatenate([ew, jnp.zeros((npad,), F32)])
    zrows = jnp.zeros((RPT, MW), F32)
    zc = jnp.zeros((RPT, CW), F32)
    onesb = jnp.ones((CHUNK, CW), F32)
    wl0 = p['gl0_W1'][:, XE + GH]
    wl1 = p['gl1_W1'][:, GH]

    pp0, pc = _sc_pass0(y0, rowp, colp, ewp, wl0, zrows, zc, onesb)
    pcn = pc[:, :N]
    h0, y1 = _k4(pp0[:, :N], pcn, nx,
                 p['gl0_W2'].T, p['gl0_b2'].reshape(1, -1),
                 p['gl1_W1'][:, :GH].T, p['gl1_b1'].reshape(1, -1))

    pp1 = _sc_pass1(y1, rowp, colp, ewp, wl1, zrows)
    res = _k5(pp1[:, :N], pcn, h0, p['gl1_W2'].T, p['gl1_b2'].reshape(1, -1),
              p['pred_W1'].T, p['pred_b1'].reshape(1, -1),
              p['pred_W2'].T, p['pred_b2'].reshape(1, -1))
    return res.reshape(1, N, PS)


# speedup vs baseline: 3.1547x; 1.0467x over previous
import functools

import jax
import jax.numpy as jnp
from jax import lax
from jax.experimental import pallas as pl
from jax.experimental.pallas import tpu as pltpu
from jax.experimental.pallas import tpu_sc as plsc

F32 = jnp.float32
I32 = jnp.int32

TW = 24
N = 10000
XE = 32
LOCE = 12
EH = 16
GH = 64
PS = 6
G = 16
E = 160000
NW = 32
CHUNK = 128
EPW = 5120
EPAD = NW * EPW
NCH = EPW // CHUNK
NPAD = 10240
RPT = NPAD // 16
MW = GH
CW = 16

ROWS = 1000
GRID = 10


def _k1_body(x_ref, loc_ref, wp_ref, wih_ref, whh_ref, b_ref, locw_ref,
             locb_ref, h_ref, w_ref, gx_ref):
    i = pl.program_id(0)
    h = jnp.zeros((ROWS, XE), F32)
    c = jnp.zeros((ROWS, XE), F32)
    wih = wih_ref[...]
    whh = whh_ref[...]
    b = b_ref[...]
    dn = (((0,), (0,)), ((), ()))
    for t in range(TW):
        xt = x_ref[:, t * 8:(t + 1) * 8]
        g = (jnp.dot(xt, wih, preferred_element_type=F32)
             + jnp.dot(h, whh, preferred_element_type=F32) + b)
        ig = jax.nn.sigmoid(g[:, :XE])
        fg = jax.nn.sigmoid(g[:, XE:2 * XE])
        gg = jnp.tanh(g[:, 2 * XE:3 * XE])
        og = jax.nn.sigmoid(g[:, 3 * XE:])
        c = fg * c + ig * gg
        h = og * jnp.tanh(c)
    h_ref[...] = h
    wp = wp_ref[...]
    ex = jnp.exp(wp - jnp.max(wp, axis=1, keepdims=True))
    w = ex / jnp.sum(ex, axis=1, keepdims=True)
    w_ref[...] = w
    loce = (jnp.dot(loc_ref[...], locw_ref[...],
                    preferred_element_type=F32) + locb_ref[...])
    xloc = jnp.concatenate([h, loce], axis=1)
    gxp = lax.dot_general(w, xloc, dn, preferred_element_type=F32)

    @pl.when(i == 0)
    def _():
        gx_ref[...] = jnp.zeros_like(gx_ref)

    gx_ref[...] += gxp


def _k1(xr, locr, wp, wih_t, whh_t, b2, locw_t, locb2):
    full = lambda s: pl.BlockSpec(s, lambda i: (0, 0))
    return pl.pallas_call(
        _k1_body,
        grid=(GRID,),
        in_specs=[
            pl.BlockSpec((ROWS, TW * 8), lambda i: (i, 0)),
            pl.BlockSpec((ROWS, 2), lambda i: (i, 0)),
            pl.BlockSpec((ROWS, G), lambda i: (i, 0)),
            full((8, 4 * XE)),
            full((XE, 4 * XE)),
            full((1, 4 * XE)),
            full((2, LOCE)),
            full((1, LOCE)),
        ],
        out_specs=[
            pl.BlockSpec((ROWS, XE), lambda i: (i, 0)),
            pl.BlockSpec((ROWS, G), lambda i: (i, 0)),
            pl.BlockSpec((G, XE + LOCE), lambda i: (0, 0)),
        ],
        out_shape=[
            jax.ShapeDtypeStruct((N, XE), F32),
            jax.ShapeDtypeStruct((N, G), F32),
            jax.ShapeDtypeStruct((G, XE + LOCE), F32),
        ],
    )(xr, locr, wp, wih_t, whh_t, b2, locw_t, locb2)


def _k2_body(gx_ref, u_ref, ue1_ref, ue2_ref, ue3_ref, si_ref, sj_ref, d_ref,
             ew_t_ref, eb_ref, a1_ref, ab1_ref, a2_ref, ab2_ref, c1_ref,
             cb1_ref, c2_ref, cb2_ref, out_ref):
    gx = gx_ref[...]
    si = si_ref[...]
    sj = sj_ref[...]
    dsel = d_ref[...]
    u0 = u_ref[0]
    u1 = u_ref[1]
    u2 = u_ref[2]
    ue = jnp.concatenate([
        ue1_ref[pl.ds(u0, 1), :],
        ue2_ref[pl.ds(u1, 1), :],
        ue3_ref[pl.ds(u2, 1), :],
    ], axis=1)
    gi = jnp.dot(si, gx, preferred_element_type=F32)
    gj = jnp.dot(sj, gx, preferred_element_type=F32)
    gin = jnp.concatenate(
        [gi, gj, jnp.broadcast_to(ue, (256, 12))], axis=1)
    ge = jnp.maximum(
        jnp.dot(gin, ew_t_ref[...], preferred_element_type=F32)
        + eb_ref[...], 0.0)
    dn = (((0,), (0,)), ((), ()))

    def node_layer(xg, w1t, b1, w2t, b2):
        xi = jnp.dot(si, xg, preferred_element_type=F32)
        m = jnp.maximum(
            jnp.dot(jnp.concatenate([xi, ge], axis=1), w1t,
                    preferred_element_type=F32) + b1, 0.0)
        ssum = lax.dot_general(sj, m, dn, preferred_element_type=F32)
        diag = jnp.dot(dsel, m, preferred_element_type=F32)
        mean = (ssum - diag) * (1.0 / 15.0)
        return jnp.maximum(
            jnp.dot(jnp.concatenate([xg, mean], axis=1), w2t,
                    preferred_element_type=F32) + b2, 0.0)

    g1 = node_layer(gx, a1_ref[...], ab1_ref[...], a2_ref[...], ab2_ref[...])
    g2 = node_layer(g1, c1_ref[...], cb1_ref[...], c2_ref[...], cb2_ref[...])
    out_ref[...] = g2


def _k2(gx, u, ue1, ue2, ue3, si, sj, dsel, ew_t, eb, a1, ab1, a2, ab2, c1,
        cb1, c2, cb2):
    args = (gx, u, ue1, ue2, ue3, si, sj, dsel, ew_t, eb, a1, ab1, a2, ab2,
            c1, cb1, c2, cb2)
    in_specs = [pl.BlockSpec(a.shape, functools.partial(lambda n: (0,) * n,
                                                        a.ndim))
                for a in args]
    in_specs[1] = pl.BlockSpec(memory_space=pltpu.SMEM)
    return pl.pallas_call(
        _k2_body,
        in_specs=in_specs,
        out_specs=pl.BlockSpec((G, GH), lambda: (0, 0)),
        out_shape=jax.ShapeDtypeStruct((G, GH), F32),
    )(*args)


_ER = E // 128
_PR = EPAD // 128


def _k0_body(r_ref, c_ref, w_ref, rp_ref, cp_ref, wp_ref):
    rp_ref[:_ER] = r_ref[...]
    rp_ref[_ER:] = jnp.zeros((_PR - _ER, 128), I32)
    cp_ref[:_ER] = c_ref[...]
    cp_ref[_ER:] = jnp.full((_PR - _ER, 128), N, I32)
    wp_ref[:_ER] = w_ref[...]
    wp_ref[_ER:] = jnp.zeros((_PR - _ER, 128), F32)


def _k0(er0, er1, ewr):
    full = lambda s: pl.BlockSpec(s, lambda: (0, 0))
    return pl.pallas_call(
        _k0_body,
        in_specs=[full((_ER, 128))] * 3,
        out_specs=[full((_PR, 128))] * 3,
        out_shape=[
            jax.ShapeDtypeStruct((_PR, 128), I32),
            jax.ShapeDtypeStruct((_PR, 128), I32),
            jax.ShapeDtypeStruct((_PR, 128), F32),
        ],
    )(er0, er1, ewr)


def _k3_body(h_ref, w_ref, gx2_ref, w1t_ref, b1_ref, nx_ref, y_ref):
    h = h_ref[...]
    gnew = jnp.dot(w_ref[...], gx2_ref[...], preferred_element_type=F32)
    nx = jnp.concatenate([h, gnew], axis=1)
    nx_ref[...] = nx
    y_ref[...] = (jnp.dot(nx, w1t_ref[...], preferred_element_type=F32)
                  + b1_ref[...])


def _k3(h, w, gx2, w1t, b1):
    full = lambda s: pl.BlockSpec(s, lambda i: (0, 0))
    return pl.pallas_call(
        _k3_body,
        grid=(GRID,),
        in_specs=[
            pl.BlockSpec((ROWS, XE), lambda i: (i, 0)),
            pl.BlockSpec((ROWS, G), lambda i: (i, 0)),
            full((G, GH)),
            full((XE + GH, GH)),
            full((1, GH)),
        ],
        out_specs=[
            pl.BlockSpec((ROWS, XE + GH), lambda i: (i, 0)),
            pl.BlockSpec((ROWS, MW), lambda i: (i, 0)),
        ],
        out_shape=[
            jax.ShapeDtypeStruct((N, XE + GH), F32),
            jax.ShapeDtypeStruct((N, MW), F32),
        ],
    )(h, w, gx2, w1t, b1)


NBUF = 4


def _make_sc_kernel(do_counts):
    def body(*refs):
        if do_counts:
            (y_hbm, row_hbm, col_hbm, ew_hbm, wl_hbm, z_hbm, zc_hbm, ones_hbm,
             out_hbm, cnt_hbm,
             r0, r1, r2, r3, c0, c1, c2, c3, m0, m1, m2, m3,
             ewall, wl_v, ones_v, sums, csum,
             ir0, ir1, ir2, ir3, ic0, ic1, ic2, ic3,
             g0, g1, g2, g3, s0, s1, s2, s3) = refs
        else:
            (y_hbm, row_hbm, col_hbm, ew_hbm, wl_hbm, z_hbm,
             out_hbm,
             r0, r1, r2, r3, c0, c1, c2, c3, m0, m1, m2, m3,
             ewall, wl_v, sums,
             ir0, ir1, ir2, ir3, ic0, ic1, ic2, ic3,
             g0, g1, g2, g3, s0, s1, s2, s3) = refs
        cid = lax.axis_index("c")
        sid = lax.axis_index("s")
        wid = sid * 2 + cid
        base = wid * EPW
        rowb = [r0, r1, r2, r3]
        colb = [c0, c1, c2, c3]
        msgb = [m0, m1, m2, m3]
        irsem = [ir0, ir1, ir2, ir3]
        icsem = [ic0, ic1, ic2, ic3]
        gsem = [g0, g1, g2, g3]
        ssem = [s0, s1, s2, s3]
        pltpu.sync_copy(ew_hbm.at[pl.ds(base, EPW)], ewall)
        pltpu.sync_copy(wl_hbm, wl_v)
        pltpu.sync_copy(z_hbm, sums.at[pl.ds(sid * RPT, RPT)])
        if do_counts:
            pltpu.sync_copy(ones_hbm, ones_v)
            pltpu.sync_copy(zc_hbm, csum.at[pl.ds(sid * RPT, RPT)])
        plsc.subcore_barrier()
        wl = [wl_v[pl.ds(j * 16, 16)] for j in range(4)]

        def idx_copy(chunk, slot):
            off = base + chunk * CHUNK
            pltpu.async_copy(row_hbm.at[pl.ds(off, CHUNK)], rowb[slot],
                             irsem[slot])
            pltpu.async_copy(col_hbm.at[pl.ds(off, CHUNK)], colb[slot],
                             icsem[slot])

        def idx_wait(slot):
            pltpu.make_async_copy(row_hbm.at[pl.ds(0, CHUNK)], rowb[slot],
                                  irsem[slot]).wait()
            pltpu.make_async_copy(col_hbm.at[pl.ds(0, CHUNK)], colb[slot],
                                  icsem[slot]).wait()

        def gather(slot):
            pltpu.async_copy(y_hbm.at[rowb[slot]], msgb[slot], gsem[slot])

        def gather_wait(slot):
            pltpu.make_async_copy(y_hbm.at[rowb[slot]], msgb[slot],
                                  gsem[slot]).wait()

        def scatter(slot):
            pltpu.async_copy(msgb[slot], sums.at[colb[slot]], ssem[slot],
                             add=True)

        def scatter_wait(slot):
            pltpu.make_async_copy(msgb[slot], sums.at[colb[slot]],
                                  ssem[slot]).wait()

        def compute(cn, b):
            def grp_body(g, c2):
                ewg = ewall[pl.ds(cn * CHUNK + g * 16, 16)]
                for l in range(16):
                    ewb = jnp.full((16,), ewg[l], F32)
                    e = g * 16 + l
                    for j in range(4):
                        v = msgb[b][e, pl.ds(j * 16, 16)]
                        msgb[b][e, pl.ds(j * 16, 16)] = jnp.maximum(
                            v + ewb * wl[j], 0.0)
                return c2

            lax.fori_loop(0, CHUNK // 16, grp_body, 0)

        idx_copy(0, 0)
        idx_copy(1, 1)
        idx_wait(0)
        gather(0)

        def outer(k, carry):
            for b in range(NBUF):
                cn = k * NBUF + b
                f = (b + 2) % NBUF
                nb = (b + 1) % NBUF
                if b >= 2:
                    scatter_wait(f)
                else:
                    @pl.when(k >= 1)
                    def _():
                        scatter_wait(f)
                idx_copy(jnp.minimum(cn + 2, NCH - 1), f)
                idx_wait(nb)
                gather(nb)
                gather_wait(b)
                compute(cn, b)
                scatter(b)
                if do_counts:
                    pltpu.sync_copy(ones_v, csum.at[colb[b]], add=True)
            return carry

        lax.fori_loop(0, NCH // NBUF, outer, 0)
        gather_wait(0)
        idx_wait(1)
        scatter_wait(2)
        scatter_wait(3)
        plsc.subcore_barrier()
        pltpu.sync_copy(sums.at[pl.ds(sid * RPT, RPT)],
                        out_hbm.at[cid, pl.ds(sid * RPT, RPT)])
        if do_counts:
            pltpu.sync_copy(csum.at[pl.ds(sid * RPT, RPT)],
                            cnt_hbm.at[cid, pl.ds(sid * RPT, RPT)])

    out_type = [jax.ShapeDtypeStruct((2, NPAD, MW), F32)]
    scratch = (
        [pltpu.VMEM((CHUNK,), I32)] * 8
        + [pltpu.VMEM((CHUNK, MW), F32)] * 4
        + [pltpu.VMEM((EPW,), F32), pltpu.VMEM((MW,), F32)]
    )
    if do_counts:
        out_type.append(jax.ShapeDtypeStruct((2, NPAD, CW), F32))
        scratch.append(pltpu.VMEM((CHUNK, CW), F32))
    scratch.append(pltpu.VMEM_SHARED((NPAD, MW), F32))
    if do_counts:
        scratch.append(pltpu.VMEM_SHARED((NPAD, CW), F32))
    scratch += [pltpu.SemaphoreType.DMA] * 16
    mesh = plsc.VectorSubcoreMesh(core_axis_name="c", subcore_axis_name="s")
    return pl.kernel(
        body,
        mesh=mesh,
        compiler_params=pltpu.CompilerParams(use_tc_tiling_on_sc=False),
        out_type=out_type if do_counts else out_type[0],
        scratch_types=scratch,
    )


_make_sc_kernel = functools.cache(_make_sc_kernel)


def _sc_pass0(*args):
    return _make_sc_kernel(True)(*args)


def _sc_pass1(*args):
    return _make_sc_kernel(False)(*args)


def _k4_body(pp_ref, pc_ref, nx_ref, w2t_ref, b2_ref, w1t_ref, b1_ref,
             h0_ref, y_ref):
    s = pp_ref[0] + pp_ref[1]
    cnt = pc_ref[0, :, :1] + pc_ref[1, :, :1]
    mean = s / jnp.maximum(cnt, 1.0)
    h0 = jnp.maximum(
        jnp.dot(jnp.concatenate([nx_ref[...], mean], axis=1), w2t_ref[...],
                preferred_element_type=F32) + b2_ref[...], 0.0)
    h0_ref[...] = h0
    y_ref[...] = (jnp.dot(h0, w1t_ref[...], preferred_element_type=F32)
                  + b1_ref[...])


def _k4(pp, pc, nx, w2t, b2, w1t, b1):
    full = lambda s: pl.BlockSpec(s, lambda i: (0, 0))
    return pl.pallas_call(
        _k4_body,
        grid=(GRID,),
        in_specs=[
            pl.BlockSpec((2, ROWS, MW), lambda i: (0, i, 0)),
            pl.BlockSpec((2, ROWS, CW), lambda i: (0, i, 0)),
            pl.BlockSpec((ROWS, XE + GH), lambda i: (i, 0)),
            full((XE + 2 * GH, GH)),
            full((1, GH)),
            full((GH, GH)),
            full((1, GH)),
        ],
        out_specs=[
            pl.BlockSpec((ROWS, GH), lambda i: (i, 0)),
            pl.BlockSpec((ROWS, MW), lambda i: (i, 0)),
        ],
        out_shape=[
            jax.ShapeDtypeStruct((N, GH), F32),
            jax.ShapeDtypeStruct((N, MW), F32),
        ],
    )(pp, pc, nx, w2t, b2, w1t, b1)


def _k5_body(pp_ref, pc_ref, h0_ref, w2t_ref, b2_ref, pw1_ref, pb1_ref,
             pw2_ref, pb2_ref, out_ref):
    s = pp_ref[0] + pp_ref[1]
    cnt = pc_ref[0, :, :1] + pc_ref[1, :, :1]
    mean = s / jnp.maximum(cnt, 1.0)
    h1 = jnp.maximum(
        jnp.dot(jnp.concatenate([h0_ref[...], mean], axis=1), w2t_ref[...],
                preferred_element_type=F32) + b2_ref[...], 0.0)
    r = jnp.maximum(
        jnp.dot(h1, pw1_ref[...], preferred_element_type=F32) + pb1_ref[...],
        0.0)
    out_ref[...] = jnp.maximum(
        jnp.dot(r, pw2_ref[...], preferred_element_type=F32) + pb2_ref[...],
        0.0)


def _k5(pp, pc, h0, w2t, b2, pw1, pb1, pw2, pb2):
    full = lambda s: pl.BlockSpec(s, lambda i: (0, 0))
    return pl.pallas_call(
        _k5_body,
        grid=(GRID,),
        in_specs=[
            pl.BlockSpec((2, ROWS, MW), lambda i: (0, i, 0)),
            pl.BlockSpec((2, ROWS, CW), lambda i: (0, i, 0)),
            pl.BlockSpec((ROWS, GH), lambda i: (i, 0)),
            full((2 * GH, GH)),
            full((1, GH)),
            full((GH, 16)),
            full((1, 16)),
            full((16, PS)),
            full((1, PS)),
        ],
        out_specs=pl.BlockSpec((ROWS, PS), lambda i: (i, 0)),
        out_shape=jax.ShapeDtypeStruct((N, PS), F32),
    )(pp, pc, h0, w2t, b2, pw1, pb1, pw2, pb2)


def kernel(x, u, edge_index, edge_w, loc, params):
    p = params
    xr = x.reshape(N, TW * 8).astype(F32)
    locr = loc.reshape(N, 2).astype(F32)
    u_i = u.reshape(3).astype(I32)
    wih_t = p['lstm_Wih'].T
    whh_t = p['lstm_Whh'].T
    b2 = (p['lstm_bih'] + p['lstm_bhh']).reshape(1, -1)
    locw_t = p['loc_W'].T
    locb2 = p['loc_b'].reshape(1, -1)

    h, w, gx = _k1(xr, locr, p['w_param'], wih_t, whh_t, b2, locw_t, locb2)

    pr = jnp.arange(256)
    si = jax.nn.one_hot(pr // G, G, dtype=F32)
    sj = jax.nn.one_hot(pr % G, G, dtype=F32)
    dsel = jax.nn.one_hot(jnp.arange(G) * (G + 1), 256, dtype=F32)
    gx2 = _k2(gx, u_i, p['uemb1'], p['uemb2'], p['uemb3'], si, sj, dsel,
              p['einf_W'].T, p['einf_b'].reshape(1, -1),
              p['gg0_W1'].T, p['gg0_b1'].reshape(1, -1),
              p['gg0_W2'].T, p['gg0_b2'].reshape(1, -1),
              p['gg1_W1'].T, p['gg1_b1'].reshape(1, -1),
              p['gg1_W2'].T, p['gg1_b2'].reshape(1, -1))

    nx, y0 = _k3(h, w, gx2, p['gl0_W1'][:, :XE + GH].T,
                 p['gl0_b1'].reshape(1, -1))

    er0 = edge_index[0, 0].astype(I32).reshape(_ER, 128)
    er1 = edge_index[0, 1].astype(I32).reshape(_ER, 128)
    ewr = edge_w.astype(F32).reshape(_ER, 128)
    rowq, colq, ewq = _k0(er0, er1, ewr)
    rowp = rowq.reshape(EPAD)
    colp = colq.reshape(EPAD)
    ewp = ewq.reshape(EPAD)
    zrows = jnp.zeros((RPT, MW), F32)
    zc = jnp.zeros((RPT, CW), F32)
    onesb = jnp.ones((CHUNK, CW), F32)
    wl0 = p['gl0_W1'][:, XE + GH]
    wl1 = p['gl1_W1'][:, GH]

    pp0, pc = _sc_pass0(y0, rowp, colp, ewp, wl0, zrows, zc, onesb)
    h0, y1 = _k4(pp0, pc, nx,
                 p['gl0_W2'].T, p['gl0_b2'].reshape(1, -1),
                 p['gl1_W1'][:, :GH].T, p['gl1_b1'].reshape(1, -1))

    pp1 = _sc_pass1(y1, rowp, colp, ewp, wl1, zrows)
    res = _k5(pp1, pc, h0, p['gl1_W2'].T, p['gl1_b2'].reshape(1, -1),
              p['pred_W1'].T, p['pred_b1'].reshape(1, -1),
              p['pred_W2'].T, p['pred_b2'].reshape(1, -1))
    return res.reshape(1, N, PS)


# speedup vs baseline: 3.3718x; 1.0688x over previous
import functools

import jax
import jax.numpy as jnp
from jax import lax
from jax.experimental import pallas as pl
from jax.experimental.pallas import tpu as pltpu
from jax.experimental.pallas import tpu_sc as plsc

F32 = jnp.float32
I32 = jnp.int32

TW = 24
N = 10000
XE = 32
LOCE = 12
EH = 16
GH = 64
PS = 6
G = 16
E = 160000
NW = 32
CHUNK = 128
EPW = 5120
EPAD = NW * EPW
NCH = EPW // CHUNK
NPAD = 10240
RPT = NPAD // 16
MW = GH
CW = 16

ROWS = 1000
GRID = 10


def _k1_body(x_ref, loc_ref, wp_ref, wih_ref, whh_ref, b_ref, locw_ref,
             locb_ref, h_ref, w_ref, gx_ref):
    h = jnp.zeros((N, XE), F32)
    c = jnp.zeros((N, XE), F32)
    wih = wih_ref[...]
    whh = whh_ref[...]
    b = b_ref[...]
    dn = (((0,), (0,)), ((), ()))
    for t in range(TW):
        xt = x_ref[:, t * 8:(t + 1) * 8]
        g = (jnp.dot(xt, wih, preferred_element_type=F32)
             + jnp.dot(h, whh, preferred_element_type=F32) + b)
        ig = jax.nn.sigmoid(g[:, :XE])
        fg = jax.nn.sigmoid(g[:, XE:2 * XE])
        gg = jnp.tanh(g[:, 2 * XE:3 * XE])
        og = jax.nn.sigmoid(g[:, 3 * XE:])
        c = fg * c + ig * gg
        h = og * jnp.tanh(c)
    h_ref[...] = h
    wp = wp_ref[...]
    ex = jnp.exp(wp - jnp.max(wp, axis=1, keepdims=True))
    w = ex / jnp.sum(ex, axis=1, keepdims=True)
    w_ref[...] = w
    loce = (jnp.dot(loc_ref[...], locw_ref[...],
                    preferred_element_type=F32) + locb_ref[...])
    xloc = jnp.concatenate([h, loce], axis=1)
    gx_ref[...] = lax.dot_general(w, xloc, dn,
                                  preferred_element_type=F32)


def _k1(xr, locr, wp, wih_t, whh_t, b2, locw_t, locb2):
    full = lambda s: pl.BlockSpec(s, lambda: (0,) * len(s))
    return pl.pallas_call(
        _k1_body,
        in_specs=[
            full((N, TW * 8)),
            full((N, 2)),
            full((N, G)),
            full((8, 4 * XE)),
            full((XE, 4 * XE)),
            full((1, 4 * XE)),
            full((2, LOCE)),
            full((1, LOCE)),
        ],
        out_specs=[
            full((N, XE)),
            full((N, G)),
            full((G, XE + LOCE)),
        ],
        out_shape=[
            jax.ShapeDtypeStruct((N, XE), F32),
            jax.ShapeDtypeStruct((N, G), F32),
            jax.ShapeDtypeStruct((G, XE + LOCE), F32),
        ],
    )(xr, locr, wp, wih_t, whh_t, b2, locw_t, locb2)


def _k2_body(gx_ref, u_ref, ue1_ref, ue2_ref, ue3_ref, si_ref, sj_ref, d_ref,
             ew_t_ref, eb_ref, a1_ref, ab1_ref, a2_ref, ab2_ref, c1_ref,
             cb1_ref, c2_ref, cb2_ref, out_ref):
    gx = gx_ref[...]
    si = si_ref[...]
    sj = sj_ref[...]
    dsel = d_ref[...]
    u0 = u_ref[0]
    u1 = u_ref[1]
    u2 = u_ref[2]
    ue = jnp.concatenate([
        ue1_ref[pl.ds(u0, 1), :],
        ue2_ref[pl.ds(u1, 1), :],
        ue3_ref[pl.ds(u2, 1), :],
    ], axis=1)
    gi = jnp.dot(si, gx, preferred_element_type=F32)
    gj = jnp.dot(sj, gx, preferred_element_type=F32)
    gin = jnp.concatenate(
        [gi, gj, jnp.broadcast_to(ue, (256, 12))], axis=1)
    ge = jnp.maximum(
        jnp.dot(gin, ew_t_ref[...], preferred_element_type=F32)
        + eb_ref[...], 0.0)
    dn = (((0,), (0,)), ((), ()))

    def node_layer(xg, w1t, b1, w2t, b2):
        xi = jnp.dot(si, xg, preferred_element_type=F32)
        m = jnp.maximum(
            jnp.dot(jnp.concatenate([xi, ge], axis=1), w1t,
                    preferred_element_type=F32) + b1, 0.0)
        ssum = lax.dot_general(sj, m, dn, preferred_element_type=F32)
        diag = jnp.dot(dsel, m, preferred_element_type=F32)
        mean = (ssum - diag) * (1.0 / 15.0)
        return jnp.maximum(
            jnp.dot(jnp.concatenate([xg, mean], axis=1), w2t,
                    preferred_element_type=F32) + b2, 0.0)

    g1 = node_layer(gx, a1_ref[...], ab1_ref[...], a2_ref[...], ab2_ref[...])
    g2 = node_layer(g1, c1_ref[...], cb1_ref[...], c2_ref[...], cb2_ref[...])
    out_ref[...] = g2


def _k2(gx, u, ue1, ue2, ue3, si, sj, dsel, ew_t, eb, a1, ab1, a2, ab2, c1,
        cb1, c2, cb2):
    args = (gx, u, ue1, ue2, ue3, si, sj, dsel, ew_t, eb, a1, ab1, a2, ab2,
            c1, cb1, c2, cb2)
    in_specs = [pl.BlockSpec(a.shape, functools.partial(lambda n: (0,) * n,
                                                        a.ndim))
                for a in args]
    in_specs[1] = pl.BlockSpec(memory_space=pltpu.SMEM)
    return pl.pallas_call(
        _k2_body,
        in_specs=in_specs,
        out_specs=pl.BlockSpec((G, GH), lambda: (0, 0)),
        out_shape=jax.ShapeDtypeStruct((G, GH), F32),
    )(*args)


_ER = E // 128
_PR = EPAD // 128


def _k0_body(r_ref, c_ref, w_ref, rp_ref, cp_ref, wp_ref):
    rp_ref[:_ER] = r_ref[...]
    rp_ref[_ER:] = jnp.zeros((_PR - _ER, 128), I32)
    cp_ref[:_ER] = c_ref[...]
    cp_ref[_ER:] = jnp.full((_PR - _ER, 128), N, I32)
    wp_ref[:_ER] = w_ref[...]
    wp_ref[_ER:] = jnp.zeros((_PR - _ER, 128), F32)


def _k0(er0, er1, ewr):
    full = lambda s: pl.BlockSpec(s, lambda: (0, 0))
    return pl.pallas_call(
        _k0_body,
        in_specs=[full((_ER, 128))] * 3,
        out_specs=[full((_PR, 128))] * 3,
        out_shape=[
            jax.ShapeDtypeStruct((_PR, 128), I32),
            jax.ShapeDtypeStruct((_PR, 128), I32),
            jax.ShapeDtypeStruct((_PR, 128), F32),
        ],
    )(er0, er1, ewr)


def _k3_body(h_ref, w_ref, gx2_ref, w1t_ref, b1_ref, nx_ref, y_ref):
    h = h_ref[...]
    gnew = jnp.dot(w_ref[...], gx2_ref[...], preferred_element_type=F32)
    nx = jnp.concatenate([h, gnew], axis=1)
    nx_ref[...] = nx
    y_ref[...] = (jnp.dot(nx, w1t_ref[...], preferred_element_type=F32)
                  + b1_ref[...])


def _k3(h, w, gx2, w1t, b1):
    full = lambda s: pl.BlockSpec(s, lambda i: (0, 0))
    return pl.pallas_call(
        _k3_body,
        grid=(GRID,),
        in_specs=[
            pl.BlockSpec((ROWS, XE), lambda i: (i, 0)),
            pl.BlockSpec((ROWS, G), lambda i: (i, 0)),
            full((G, GH)),
            full((XE + GH, GH)),
            full((1, GH)),
        ],
        out_specs=[
            pl.BlockSpec((ROWS, XE + GH), lambda i: (i, 0)),
            pl.BlockSpec((ROWS, MW), lambda i: (i, 0)),
        ],
        out_shape=[
            jax.ShapeDtypeStruct((N, XE + GH), F32),
            jax.ShapeDtypeStruct((N, MW), F32),
        ],
    )(h, w, gx2, w1t, b1)


NBUF = 4
NCH0 = 56
NCH1 = 24
CH0T = 16 * NCH0
EWB = NCH0 * CHUNK


def _make_sc_kernel(do_counts):
    def body(*refs):
        if do_counts:
            (y_hbm, row_hbm, col_hbm, ew_hbm, wl_hbm, z_hbm, zc_hbm, ones_hbm,
             out_hbm, cnt_hbm,
             r0, r1, r2, r3, c0, c1, c2, c3, m0, m1, m2, m3,
             ewall, wl_v, ones_v, sums, csum,
             ir0, ir1, ir2, ir3, ic0, ic1, ic2, ic3,
             g0, g1, g2, g3, s0, s1, s2, s3) = refs
        else:
            (y_hbm, row_hbm, col_hbm, ew_hbm, wl_hbm, z_hbm,
             out_hbm,
             r0, r1, r2, r3, c0, c1, c2, c3, m0, m1, m2, m3,
             ewall, wl_v, sums,
             ir0, ir1, ir2, ir3, ic0, ic1, ic2, ic3,
             g0, g1, g2, g3, s0, s1, s2, s3) = refs
        cid = lax.axis_index("c")
        sid = lax.axis_index("s")
        cbase = jnp.where(cid == 0, sid * NCH0, CH0T + sid * NCH1)
        nch = jnp.where(cid == 0, NCH0, NCH1)
        base = cbase * CHUNK
        ewbase = jnp.minimum(base, EPAD - EWB)
        ewoff = base - ewbase
        rowb = [r0, r1, r2, r3]
        colb = [c0, c1, c2, c3]
        msgb = [m0, m1, m2, m3]
        irsem = [ir0, ir1, ir2, ir3]
        icsem = [ic0, ic1, ic2, ic3]
        gsem = [g0, g1, g2, g3]
        ssem = [s0, s1, s2, s3]
        pltpu.sync_copy(ew_hbm.at[pl.ds(ewbase, EWB)], ewall)
        pltpu.sync_copy(wl_hbm, wl_v)
        pltpu.sync_copy(z_hbm, sums.at[pl.ds(sid * RPT, RPT)])
        if do_counts:
            pltpu.sync_copy(ones_hbm, ones_v)
            pltpu.sync_copy(zc_hbm, csum.at[pl.ds(sid * RPT, RPT)])
        plsc.subcore_barrier()
        wl = [wl_v[pl.ds(j * 16, 16)] for j in range(4)]

        def idx_copy(chunk, slot):
            off = base + chunk * CHUNK
            pltpu.async_copy(row_hbm.at[pl.ds(off, CHUNK)], rowb[slot],
                             irsem[slot])
            pltpu.async_copy(col_hbm.at[pl.ds(off, CHUNK)], colb[slot],
                             icsem[slot])

        def idx_wait(slot):
            pltpu.make_async_copy(row_hbm.at[pl.ds(0, CHUNK)], rowb[slot],
                                  irsem[slot]).wait()
            pltpu.make_async_copy(col_hbm.at[pl.ds(0, CHUNK)], colb[slot],
                                  icsem[slot]).wait()

        def gather(slot):
            pltpu.async_copy(y_hbm.at[rowb[slot]], msgb[slot], gsem[slot])

        def gather_wait(slot):
            pltpu.make_async_copy(y_hbm.at[rowb[slot]], msgb[slot],
                                  gsem[slot]).wait()

        def scatter(slot):
            pltpu.async_copy(msgb[slot], sums.at[colb[slot]], ssem[slot],
                             add=True)

        def scatter_wait(slot):
            pltpu.make_async_copy(msgb[slot], sums.at[colb[slot]],
                                  ssem[slot]).wait()

        def compute(cn, b):
            def grp_body(g, c2):
                ewg = ewall[pl.ds(ewoff + cn * CHUNK + g * 16, 16)]
                for l in range(16):
                    ewb = jnp.full((16,), ewg[l], F32)
                    e = g * 16 + l
                    for j in range(4):
                        v = msgb[b][e, pl.ds(j * 16, 16)]
                        msgb[b][e, pl.ds(j * 16, 16)] = jnp.maximum(
                            v + ewb * wl[j], 0.0)
                return c2

            lax.fori_loop(0, CHUNK // 16, grp_body, 0)

        idx_copy(0, 0)
        idx_copy(1, 1)
        idx_wait(0)
        gather(0)

        def outer(k, carry):
            for b in range(NBUF):
                cn = k * NBUF + b
                f = (b + 2) % NBUF
                nb = (b + 1) % NBUF
                if b >= 2:
                    scatter_wait(f)
                else:
                    @pl.when(k >= 1)
                    def _():
                        scatter_wait(f)
                idx_copy(jnp.minimum(cn + 2, nch - 1), f)
                idx_wait(nb)
                gather(nb)
                gather_wait(b)
                compute(cn, b)
                scatter(b)
                if do_counts:
                    pltpu.sync_copy(ones_v, csum.at[colb[b]], add=True)
            return carry

        lax.fori_loop(0, nch // NBUF, outer, 0)
        gather_wait(0)
        idx_wait(1)
        scatter_wait(2)
        scatter_wait(3)
        plsc.subcore_barrier()
        pltpu.sync_copy(sums.at[pl.ds(sid * RPT, RPT)],
                        out_hbm.at[cid, pl.ds(sid * RPT, RPT)])
        if do_counts:
            pltpu.sync_copy(csum.at[pl.ds(sid * RPT, RPT)],
                            cnt_hbm.at[cid, pl.ds(sid * RPT, RPT)])

    out_type = [jax.ShapeDtypeStruct((2, NPAD, MW), F32)]
    scratch = (
        [pltpu.VMEM((CHUNK,), I32)] * 8
        + [pltpu.VMEM((CHUNK, MW), F32)] * 4
        + [pltpu.VMEM((EWB,), F32), pltpu.VMEM((MW,), F32)]
    )
    if do_counts:
        out_type.append(jax.ShapeDtypeStruct((2, NPAD, CW), F32))
        scratch.append(pltpu.VMEM((CHUNK, CW), F32))
    scratch.append(pltpu.VMEM_SHARED((NPAD, MW), F32))
    if do_counts:
        scratch.append(pltpu.VMEM_SHARED((NPAD, CW), F32))
    scratch += [pltpu.SemaphoreType.DMA] * 16
    mesh = plsc.VectorSubcoreMesh(core_axis_name="c", subcore_axis_name="s")
    return pl.kernel(
        body,
        mesh=mesh,
        compiler_params=pltpu.CompilerParams(use_tc_tiling_on_sc=False),
        out_type=out_type if do_counts else out_type[0],
        scratch_types=scratch,
    )


_make_sc_kernel = functools.cache(_make_sc_kernel)


def _sc_pass0(*args):
    return _make_sc_kernel(True)(*args)


def _sc_pass1(*args):
    return _make_sc_kernel(False)(*args)


def _k4_body(pp_ref, pc_ref, nx_ref, w2t_ref, b2_ref, w1t_ref, b1_ref,
             h0_ref, y_ref):
    s = pp_ref[0] + pp_ref[1]
    cnt = pc_ref[0, :, :1] + pc_ref[1, :, :1]
    mean = s / jnp.maximum(cnt, 1.0)
    h0 = jnp.maximum(
        jnp.dot(jnp.concatenate([nx_ref[...], mean], axis=1), w2t_ref[...],
                preferred_element_type=F32) + b2_ref[...], 0.0)
    h0_ref[...] = h0
    y_ref[...] = (jnp.dot(h0, w1t_ref[...], preferred_element_type=F32)
                  + b1_ref[...])


def _k4(pp, pc, nx, w2t, b2, w1t, b1):
    full = lambda s: pl.BlockSpec(s, lambda i: (0, 0))
    return pl.pallas_call(
        _k4_body,
        grid=(GRID,),
        in_specs=[
            pl.BlockSpec((2, ROWS, MW), lambda i: (0, i, 0)),
            pl.BlockSpec((2, ROWS, CW), lambda i: (0, i, 0)),
            pl.BlockSpec((ROWS, XE + GH), lambda i: (i, 0)),
            full((XE + 2 * GH, GH)),
            full((1, GH)),
            full((GH, GH)),
            full((1, GH)),
        ],
        out_specs=[
            pl.BlockSpec((ROWS, GH), lambda i: (i, 0)),
            pl.BlockSpec((ROWS, MW), lambda i: (i, 0)),
        ],
        out_shape=[
            jax.ShapeDtypeStruct((N, GH), F32),
            jax.ShapeDtypeStruct((N, MW), F32),
        ],
    )(pp, pc, nx, w2t, b2, w1t, b1)


def _k5_body(pp_ref, pc_ref, h0_ref, w2t_ref, b2_ref, pw1_ref, pb1_ref,
             pw2_ref, pb2_ref, out_ref):
    s = pp_ref[0] + pp_ref[1]
    cnt = pc_ref[0, :, :1] + pc_ref[1, :, :1]
    mean = s / jnp.maximum(cnt, 1.0)
    h1 = jnp.maximum(
        jnp.dot(jnp.concatenate([h0_ref[...], mean], axis=1), w2t_ref[...],
                preferred_element_type=F32) + b2_ref[...], 0.0)
    r = jnp.maximum(
        jnp.dot(h1, pw1_ref[...], preferred_element_type=F32) + pb1_ref[...],
        0.0)
    out_ref[...] = jnp.maximum(
        jnp.dot(r, pw2_ref[...], preferred_element_type=F32) + pb2_ref[...],
        0.0)


def _k5(pp, pc, h0, w2t, b2, pw1, pb1, pw2, pb2):
    full = lambda s: pl.BlockSpec(s, lambda i: (0, 0))
    return pl.pallas_call(
        _k5_body,
        grid=(GRID,),
        in_specs=[
            pl.BlockSpec((2, ROWS, MW), lambda i: (0, i, 0)),
            pl.BlockSpec((2, ROWS, CW), lambda i: (0, i, 0)),
            pl.BlockSpec((ROWS, GH), lambda i: (i, 0)),
            full((2 * GH, GH)),
            full((1, GH)),
            full((GH, 16)),
            full((1, 16)),
            full((16, PS)),
            full((1, PS)),
        ],
        out_specs=pl.BlockSpec((ROWS, PS), lambda i: (i, 0)),
        out_shape=jax.ShapeDtypeStruct((N, PS), F32),
    )(pp, pc, h0, w2t, b2, pw1, pb1, pw2, pb2)


def kernel(x, u, edge_index, edge_w, loc, params):
    p = params
    xr = x.reshape(N, TW * 8).astype(F32)
    locr = loc.reshape(N, 2).astype(F32)
    u_i = u.reshape(3).astype(I32)
    wih_t = p['lstm_Wih'].T
    whh_t = p['lstm_Whh'].T
    b2 = (p['lstm_bih'] + p['lstm_bhh']).reshape(1, -1)
    locw_t = p['loc_W'].T
    locb2 = p['loc_b'].reshape(1, -1)

    h, w, gx = _k1(xr, locr, p['w_param'], wih_t, whh_t, b2, locw_t, locb2)

    pr = jnp.arange(256)
    si = jax.nn.one_hot(pr // G, G, dtype=F32)
    sj = jax.nn.one_hot(pr % G, G, dtype=F32)
    dsel = jax.nn.one_hot(jnp.arange(G) * (G + 1), 256, dtype=F32)
    gx2 = _k2(gx, u_i, p['uemb1'], p['uemb2'], p['uemb3'], si, sj, dsel,
              p['einf_W'].T, p['einf_b'].reshape(1, -1),
              p['gg0_W1'].T, p['gg0_b1'].reshape(1, -1),
              p['gg0_W2'].T, p['gg0_b2'].reshape(1, -1),
              p['gg1_W1'].T, p['gg1_b1'].reshape(1, -1),
              p['gg1_W2'].T, p['gg1_b2'].reshape(1, -1))

    nx, y0 = _k3(h, w, gx2, p['gl0_W1'][:, :XE + GH].T,
                 p['gl0_b1'].reshape(1, -1))

    er0 = edge_index[0, 0].astype(I32).reshape(_ER, 128)
    er1 = edge_index[0, 1].astype(I32).reshape(_ER, 128)
    ewr = edge_w.astype(F32).reshape(_ER, 128)
    rowq, colq, ewq = _k0(er0, er1, ewr)
    rowp = rowq.reshape(EPAD)
    colp = colq.reshape(EPAD)
    ewp = ewq.reshape(EPAD)
    zrows = jnp.zeros((RPT, MW), F32)
    zc = jnp.zeros((RPT, CW), F32)
    onesb = jnp.ones((CHUNK, CW), F32)
    wl0 = p['gl0_W1'][:, XE + GH]
    wl1 = p['gl1_W1'][:, GH]

    pp0, pc = _sc_pass0(y0, rowp, colp, ewp, wl0, zrows, zc, onesb)
    h0, y1 = _k4(pp0, pc, nx,
                 p['gl0_W2'].T, p['gl0_b2'].reshape(1, -1),
                 p['gl1_W1'][:, :GH].T, p['gl1_b1'].reshape(1, -1))

    pp1 = _sc_pass1(y1, rowp, colp, ewp, wl1, zrows)
    res = _k5(pp1, pc, h0, p['gl1_W2'].T, p['gl1_b2'].reshape(1, -1),
              p['pred_W1'].T, p['pred_b1'].reshape(1, -1),
              p['pred_W2'].T, p['pred_b2'].reshape(1, -1))
    return res.reshape(1, N, PS)
